# baseline (speedup 1.0000x reference)
import jax
import jax.numpy as jnp
from jax import lax
from jax.experimental import pallas as pl
from jax.experimental.pallas import tpu as pltpu
from jax.experimental.pallas import tpu_sc as plsc

N = 10000
E = 160000
D = 256
ED = 16

RB = 2000
EB = 800

NC = 2
NS = 16
NW = NC * NS

GK = 128
GKG = 112
G_PER = E // NW
G_FULL = G_PER // GKG
G_LAST = (G_PER - GKG) // 8 * 8

S_PER = E // NS
S_FULL = S_PER // GK
S_TAIL = S_PER - S_FULL * GK

DH = D // 2
TROW = 624


def _mm_stats_body(x_ref, w_ref, b_ref, y_ref, st_ref):
    i = pl.program_id(0)
    y = jnp.dot(x_ref[...], w_ref[...], preferred_element_type=jnp.float32)
    y = y + b_ref[...]
    y_ref[...] = y
    s1 = jnp.sum(y, axis=0)
    s2 = jnp.sum(y * y, axis=0)
    rows = lax.broadcasted_iota(jnp.int32, (8, D), 0)
    upd = jnp.where(rows == 0, s1[None, :], 0.0) + jnp.where(rows == 1, s2[None, :], 0.0)

    @pl.when(i == 0)
    def _():
        st_ref[...] = jnp.zeros_like(st_ref)

    st_ref[...] += upd


def _mm_stats(node_attr, W0, b0):
    return pl.pallas_call(
        _mm_stats_body,
        grid=(N // RB,),
        in_specs=[
            pl.BlockSpec((RB, D), lambda i: (i, 0)),
            pl.BlockSpec((D, D), lambda i: (0, 0)),
            pl.BlockSpec((1, D), lambda i: (0, 0)),
        ],
        out_specs=[
            pl.BlockSpec((RB, D), lambda i: (i, 0)),
            pl.BlockSpec((8, D), lambda i: (0, 0)),
        ],
        out_shape=[
            jax.ShapeDtypeStruct((N, D), jnp.float32),
            jax.ShapeDtypeStruct((8, D), jnp.float32),
        ],
    )(node_attr, W0, b0.reshape(1, D))


def _proj_common(x, wt_ref, wu_ref, bt_ref, x_ref, t_ref, u_ref):
    x_ref[...] = x
    t = jnp.dot(x, wt_ref[...], preferred_element_type=jnp.float32) + bt_ref[...]
    u = jnp.dot(x, wu_ref[...], preferred_element_type=jnp.float32)
    t_ref[...] = t.astype(jnp.bfloat16)
    u_ref[...] = u.astype(jnp.bfloat16)


def _proj0_body(y_ref, st_ref, g_ref, be_ref, wt_ref, wu_ref, bt_ref,
                x_ref, t_ref, u_ref):
    mean = st_ref[0:1, :] / N
    var = st_ref[1:2, :] / N - mean * mean
    inv = g_ref[...] * lax.rsqrt(var + 1e-5)
    x = jnp.maximum((y_ref[...] - mean) * inv + be_ref[...], 0.0)
    _proj_common(x, wt_ref, wu_ref, bt_ref, x_ref, t_ref, u_ref)


def _proj0(y, st, gamma, beta, W_T, W_U, b_T):
    return pl.pallas_call(
        _proj0_body,
        grid=(N // RB,),
        in_specs=[
            pl.BlockSpec((RB, D), lambda i: (i, 0)),
            pl.BlockSpec((8, D), lambda i: (0, 0)),
            pl.BlockSpec((1, D), lambda i: (0, 0)),
            pl.BlockSpec((1, D), lambda i: (0, 0)),
            pl.BlockSpec((D, 2 * D), lambda i: (0, 0)),
            pl.BlockSpec((D, 2 * D), lambda i: (0, 0)),
            pl.BlockSpec((1, 2 * D), lambda i: (0, 0)),
        ],
        out_specs=[
            pl.BlockSpec((RB, D), lambda i: (i, 0)),
            pl.BlockSpec((RB, 2 * D), lambda i: (i, 0)),
            pl.BlockSpec((RB, 2 * D), lambda i: (i, 0)),
        ],
        out_shape=[
            jax.ShapeDtypeStruct((N, D), jnp.float32),
            jax.ShapeDtypeStruct((N, 2 * D), jnp.bfloat16),
            jax.ShapeDtypeStruct((N, 2 * D), jnp.bfloat16),
        ],
    )(y, st, gamma.reshape(1, D), beta.reshape(1, D), W_T, W_U, b_T)


def _proju_body(a_ref, xp_ref, wt_ref, wu_ref, bt_ref, x_ref, t_ref, u_ref):
    a = jnp.concatenate([a_ref[0], a_ref[1]], axis=1)
    x = jnp.maximum(a + xp_ref[...], 0.0)
    _proj_common(x, wt_ref, wu_ref, bt_ref, x_ref, t_ref, u_ref)


def _proju(aggr2, xp, W_T, W_U, b_T):
    return pl.pallas_call(
        _proju_body,
        grid=(N // RB,),
        in_specs=[
            pl.BlockSpec((2, RB, DH), lambda i: (0, i, 0)),
            pl.BlockSpec((RB, D), lambda i: (i, 0)),
            pl.BlockSpec((D, 2 * D), lambda i: (0, 0)),
            pl.BlockSpec((D, 2 * D), lambda i: (0, 0)),
            pl.BlockSpec((1, 2 * D), lambda i: (0, 0)),
        ],
        out_specs=[
            pl.BlockSpec((RB, D), lambda i: (i, 0)),
            pl.BlockSpec((RB, 2 * D), lambda i: (i, 0)),
            pl.BlockSpec((RB, 2 * D), lambda i: (i, 0)),
        ],
        out_shape=[
            jax.ShapeDtypeStruct((N, D), jnp.float32),
            jax.ShapeDtypeStruct((N, 2 * D), jnp.bfloat16),
            jax.ShapeDtypeStruct((N, 2 * D), jnp.bfloat16),
        ],
    )(aggr2, xp, W_T, W_U, b_T)


def _edge_body(gt_ref, gu_ref, ea_ref, we_ref, m_ref):
    ep = jnp.dot(ea_ref[...], we_ref[...], preferred_element_type=jnp.float32)
    gz = gt_ref[...].astype(jnp.float32) + gu_ref[...].astype(jnp.float32) + ep
    f = gz[:, :D]
    s = gz[:, D:]
    sig = 1.0 / (1.0 + jnp.exp(-f))
    sp = jnp.maximum(s, 0.0) + jnp.log(1.0 + jnp.exp(-jnp.abs(s)))
    msg = sig * sp
    m_ref[...] = jnp.stack([msg[:, :DH], msg[:, DH:]], axis=0)


def _edge_stage(GT, GU, edge_attr, W_E):
    return pl.pallas_call(
        _edge_body,
        grid=(E // EB,),
        in_specs=[
            pl.BlockSpec((EB, 2 * D), lambda i: (i, 0)),
            pl.BlockSpec((EB, 2 * D), lambda i: (i, 0)),
            pl.BlockSpec((EB, ED), lambda i: (i, 0)),
            pl.BlockSpec((ED, 2 * D), lambda i: (0, 0)),
        ],
        out_specs=pl.BlockSpec((2, EB, DH), lambda i: (0, i, 0)),
        out_shape=jax.ShapeDtypeStruct((2, E, DH), jnp.float32),
    )(GT, GU, edge_attr, W_E)


def _update_body(a_ref, x_ref, o_ref):
    a = jnp.concatenate([a_ref[0], a_ref[1]], axis=1)
    o_ref[...] = jnp.maximum(a + x_ref[...], 0.0)


def _update(aggr2, x):
    return pl.pallas_call(
        _update_body,
        grid=(N // RB,),
        in_specs=[
            pl.BlockSpec((2, RB, DH), lambda i: (0, i, 0)),
            pl.BlockSpec((RB, D), lambda i: (i, 0)),
        ],
        out_specs=pl.BlockSpec((RB, D), lambda i: (i, 0)),
        out_shape=jax.ShapeDtypeStruct((N, D), jnp.float32),
    )(aggr2, x)


def _gather_body(t_hbm, u_hbm, dst_hbm, src_hbm, gt_hbm, gu_hbm,
                 dbufs, sbufs, tbufs, ubufs, semt, semu, semwt, semwu):
    wid = lax.axis_index("s") * NC + lax.axis_index("c")
    base = wid * G_PER

    def start(j, b, wait_prev):
        off = base + j * GKG
        pltpu.sync_copy(dst_hbm.at[pl.ds(off, GKG)], dbufs[b])
        pltpu.sync_copy(src_hbm.at[pl.ds(off, GKG)], sbufs[b])
        if wait_prev:
            pltpu.make_async_copy(tbufs[b], gt_hbm.at[pl.ds(off, GKG)],
                                  semwt[b]).wait()
            pltpu.make_async_copy(ubufs[b], gu_hbm.at[pl.ds(off, GKG)],
                                  semwu[b]).wait()
        pltpu.async_copy(t_hbm.at[dbufs[b]], tbufs[b], semt[b])
        pltpu.async_copy(u_hbm.at[sbufs[b]], ubufs[b], semu[b])

    def finish(j, b):
        off = base + j * GKG
        pltpu.make_async_copy(t_hbm.at[dbufs[b]], tbufs[b], semt[b]).wait()
        pltpu.make_async_copy(u_hbm.at[sbufs[b]], ubufs[b], semu[b]).wait()
        pltpu.async_copy(tbufs[b], gt_hbm.at[pl.ds(off, GKG)], semwt[b])
        pltpu.async_copy(ubufs[b], gu_hbm.at[pl.ds(off, GKG)], semwu[b])

    start(0, 0, False)
    start(1, 1, False)

    def pair(p, _):
        j0 = 2 * p
        finish(j0, 0)
        start(j0 + 2, 0, True)
        finish(j0 + 1, 1)
        start(j0 + 3, 1, True)
        return 0

    lax.fori_loop(0, G_FULL // 2 - 1, pair, 0)
    finish(G_FULL - 2, 0)
    finish(G_FULL - 1, 1)

    off = base + G_LAST
    pltpu.sync_copy(dst_hbm.at[pl.ds(off, GKG)], dbufs[0])
    pltpu.sync_copy(src_hbm.at[pl.ds(off, GKG)], sbufs[0])
    pltpu.make_async_copy(tbufs[0], gt_hbm.at[pl.ds(off, GKG)], semwt[0]).wait()
    pltpu.make_async_copy(ubufs[0], gu_hbm.at[pl.ds(off, GKG)], semwu[0]).wait()
    pltpu.make_async_copy(tbufs[1], gt_hbm.at[pl.ds(off, GKG)], semwt[1]).wait()
    pltpu.make_async_copy(ubufs[1], gu_hbm.at[pl.ds(off, GKG)], semwu[1]).wait()
    pltpu.async_copy(t_hbm.at[dbufs[0]], tbufs[0], semt[0]).wait()
    pltpu.async_copy(u_hbm.at[sbufs[0]], ubufs[0], semu[0]).wait()
    pltpu.sync_copy(tbufs[0], gt_hbm.at[pl.ds(off, GKG)])
    pltpu.sync_copy(ubufs[0], gu_hbm.at[pl.ds(off, GKG)])


def _gather(T, U, dst, src):
    mesh = plsc.VectorSubcoreMesh(core_axis_name="c", subcore_axis_name="s")
    k = pl.kernel(
        _gather_body,
        out_type=[
            jax.ShapeDtypeStruct((E, D), jnp.int32),
            jax.ShapeDtypeStruct((E, D), jnp.int32),
        ],
        mesh=mesh,
        scratch_types=[
            [pltpu.VMEM((GKG,), jnp.int32)] * 2,
            [pltpu.VMEM((GKG,), jnp.int32)] * 2,
            [pltpu.VMEM((GKG, D), jnp.int32)] * 2,
            [pltpu.VMEM((GKG, D), jnp.int32)] * 2,
            [pltpu.SemaphoreType.DMA] * 2,
            [pltpu.SemaphoreType.DMA] * 2,
            [pltpu.SemaphoreType.DMA] * 2,
            [pltpu.SemaphoreType.DMA] * 2,
        ],
    )
    return k(T, U, dst, src)


def _scatter_body(m_hbm, dst_hbm, a_hbm, spbuf, mbufs, ibufs, mbuf_t, ibuf_t, sems):
    c = lax.axis_index("c")
    sid = lax.axis_index("s")
    mbuf = mbufs[0]

    def zrow(i, _):
        r = i // (DH // 16)
        k = i % (DH // 16)
        mbuf[r, pl.ds(k * 16, 16)] = jnp.zeros((16,), jnp.float32)
        return 0

    lax.fori_loop(0, GK * (DH // 16), zrow, 0)

    start = sid * TROW
    for t in range(TROW // GK):
        pltpu.sync_copy(mbuf, spbuf.at[pl.ds(start + t * GK, GK)])
    pltpu.sync_copy(mbuf.at[pl.ds(0, TROW - (TROW // GK) * GK)],
                    spbuf.at[pl.ds(start + (TROW // GK) * GK,
                                   TROW - (TROW // GK) * GK)])

    @pl.when(sid == NS - 1)
    def _():
        pltpu.sync_copy(mbuf.at[pl.ds(0, N - NS * TROW)],
                        spbuf.at[pl.ds(NS * TROW, N - NS * TROW)])

    plsc.subcore_barrier()

    base = sid * S_PER

    def start(j, b):
        off = base + j * GK
        pltpu.sync_copy(dst_hbm.at[pl.ds(off, GK)], ibufs[b])
        pltpu.async_copy(m_hbm.at[c, pl.ds(off, GK)], mbufs[b], sems[b])

    def finish(j, b):
        off = base + j * GK
        pltpu.make_async_copy(m_hbm.at[c, pl.ds(off, GK)], mbufs[b],
                              sems[b]).wait()
        pltpu.sync_copy(mbufs[b], spbuf.at[ibufs[b]], add=True)

    start(0, 0)
    start(1, 1)

    def pair(p, _):
        j0 = 2 * p
        finish(j0, 0)
        start(j0 + 2, 0)
        finish(j0 + 1, 1)
        start(j0 + 3, 1)
        return 0

    lax.fori_loop(0, S_FULL // 2 - 1, pair, 0)
    finish(S_FULL - 2, 0)
    finish(S_FULL - 1, 1)

    off = base + S_FULL * GK
    pltpu.sync_copy(dst_hbm.at[pl.ds(off, S_TAIL)], ibuf_t)
    pltpu.sync_copy(m_hbm.at[c, pl.ds(off, S_TAIL)], mbuf_t)
    pltpu.sync_copy(mbuf_t, spbuf.at[ibuf_t], add=True)

    plsc.subcore_barrier()
    pltpu.sync_copy(spbuf.at[pl.ds(sid * TROW, TROW)],
                    a_hbm.at[c, pl.ds(sid * TROW, TROW)])

    @pl.when(sid == NS - 1)
    def _():
        pltpu.sync_copy(spbuf.at[pl.ds(NS * TROW, N - NS * TROW)],
                        a_hbm.at[c, pl.ds(NS * TROW, N - NS * TROW)])


def _scatter(msg2, dst):
    mesh = plsc.VectorSubcoreMesh(core_axis_name="c", subcore_axis_name="s")
    k = pl.kernel(
        _scatter_body,
        out_type=jax.ShapeDtypeStruct((2, N, DH), jnp.float32),
        mesh=mesh,
        scratch_types=[
            pltpu.VMEM_SHARED((N, DH), jnp.float32),
            [pltpu.VMEM((GK, DH), jnp.float32)] * 2,
            [pltpu.VMEM((GK,), jnp.int32)] * 2,
            pltpu.VMEM((S_TAIL, DH), jnp.float32),
            pltpu.VMEM((S_TAIL,), jnp.int32),
            [pltpu.SemaphoreType.DMA] * 2,
        ],
    )
    return k(msg2, dst)


def kernel(node_attr, edge_index, edge_attr, W0, b0, gamma, beta, Wf, bf, Ws, bs):
    src = edge_index[0]
    dst = edge_index[1]

    y, st = _mm_stats(node_attr, W0, b0)

    L = Wf.shape[0]
    x = None
    aggr2 = None
    for l in range(L):
        W_T = jnp.concatenate([Wf[l, :D], Ws[l, :D]], axis=1)
        W_U = jnp.concatenate([Wf[l, D:2 * D], Ws[l, D:2 * D]], axis=1)
        W_E = jnp.concatenate([Wf[l, 2 * D:], Ws[l, 2 * D:]], axis=1)
        b_T = jnp.concatenate([bf[l], bs[l]]).reshape(1, 2 * D)
        if l == 0:
            x, T, U = _proj0(y, st, gamma, beta, W_T, W_U, b_T)
        else:
            x, T, U = _proju(aggr2, x, W_T, W_U, b_T)
        Ti = lax.bitcast_convert_type(T.reshape(N, D, 2), jnp.int32)
        Ui = lax.bitcast_convert_type(U.reshape(N, D, 2), jnp.int32)
        GTi, GUi = _gather(Ti, Ui, dst, src)
        GT = lax.bitcast_convert_type(GTi, jnp.bfloat16).reshape(E, 2 * D)
        GU = lax.bitcast_convert_type(GUi, jnp.bfloat16).reshape(E, 2 * D)
        msg2 = _edge_stage(GT, GU, edge_attr, W_E)
        aggr2 = _scatter(msg2, dst)
    return _update(aggr2, x)


# speedup vs baseline: 5.3118x; 5.3118x over previous
import jax
import jax.numpy as jnp
from jax import lax
from jax.experimental import pallas as pl
from jax.experimental.pallas import tpu as pltpu
from jax.experimental.pallas import tpu_sc as plsc

N = 10000
E = 160000
D = 256
ED = 16

RB = 2000
EB = 800

NC = 2
NS = 16
NW = NC * NS

GK = 128
GKG = 112
G_PER = E // NW
G_FULL = G_PER // GKG
G_LAST = (G_PER - GKG) // 8 * 8

S_PER = E // NS
S_FULL = S_PER // GK
S_TAIL = S_PER - S_FULL * GK

DH = D // 2
TROW = 624


def _mm_stats_body(x_ref, w_ref, b_ref, y_ref, st_ref):
    i = pl.program_id(0)
    y = jnp.dot(x_ref[...], w_ref[...], preferred_element_type=jnp.float32)
    y = y + b_ref[...]
    y_ref[...] = y
    s1 = jnp.sum(y, axis=0)
    s2 = jnp.sum(y * y, axis=0)
    rows = lax.broadcasted_iota(jnp.int32, (8, D), 0)
    upd = jnp.where(rows == 0, s1[None, :], 0.0) + jnp.where(rows == 1, s2[None, :], 0.0)

    @pl.when(i == 0)
    def _():
        st_ref[...] = jnp.zeros_like(st_ref)

    st_ref[...] += upd


def _mm_stats(node_attr, W0, b0):
    return pl.pallas_call(
        _mm_stats_body,
        grid=(N // RB,),
        in_specs=[
            pl.BlockSpec((RB, D), lambda i: (i, 0)),
            pl.BlockSpec((D, D), lambda i: (0, 0)),
            pl.BlockSpec((1, D), lambda i: (0, 0)),
        ],
        out_specs=[
            pl.BlockSpec((RB, D), lambda i: (i, 0)),
            pl.BlockSpec((8, D), lambda i: (0, 0)),
        ],
        out_shape=[
            jax.ShapeDtypeStruct((N, D), jnp.float32),
            jax.ShapeDtypeStruct((8, D), jnp.float32),
        ],
    )(node_attr, W0, b0.reshape(1, D))


def _pack_bf16_pair(lo, hi):
    ul = lax.bitcast_convert_type(lo, jnp.int32)
    uh = lax.bitcast_convert_type(hi, jnp.int32)
    ul = ul + 0x7FFF + ((ul >> 16) & 1)
    uh = uh + 0x7FFF + ((uh >> 16) & 1)
    return ((ul >> 16) & 0xFFFF) | (uh & jnp.int32(-65536))


def _unpack_bf16_pair(p):
    lo = lax.bitcast_convert_type(p << 16, jnp.float32)
    hi = lax.bitcast_convert_type(p & jnp.int32(-65536), jnp.float32)
    return lo, hi


def _proj_common(x, wt_ref, wu_ref, bt_ref, x_ref, t_ref, u_ref):
    x_ref[...] = x
    t = jnp.dot(x, wt_ref[...], preferred_element_type=jnp.float32) + bt_ref[...]
    u = jnp.dot(x, wu_ref[...], preferred_element_type=jnp.float32)
    t_ref[...] = _pack_bf16_pair(t[:, :D], t[:, D:])
    u_ref[...] = _pack_bf16_pair(u[:, :D], u[:, D:])


def _proj0_body(y_ref, st_ref, g_ref, be_ref, wt_ref, wu_ref, bt_ref,
                x_ref, t_ref, u_ref):
    mean = st_ref[0:1, :] / N
    var = st_ref[1:2, :] / N - mean * mean
    inv = g_ref[...] * lax.rsqrt(var + 1e-5)
    x = jnp.maximum((y_ref[...] - mean) * inv + be_ref[...], 0.0)
    _proj_common(x, wt_ref, wu_ref, bt_ref, x_ref, t_ref, u_ref)


def _proj0(y, st, gamma, beta, W_T, W_U, b_T):
    return pl.pallas_call(
        _proj0_body,
        grid=(N // RB,),
        in_specs=[
            pl.BlockSpec((RB, D), lambda i: (i, 0)),
            pl.BlockSpec((8, D), lambda i: (0, 0)),
            pl.BlockSpec((1, D), lambda i: (0, 0)),
            pl.BlockSpec((1, D), lambda i: (0, 0)),
            pl.BlockSpec((D, 2 * D), lambda i: (0, 0)),
            pl.BlockSpec((D, 2 * D), lambda i: (0, 0)),
            pl.BlockSpec((1, 2 * D), lambda i: (0, 0)),
        ],
        out_specs=[
            pl.BlockSpec((RB, D), lambda i: (i, 0)),
            pl.BlockSpec((RB, D), lambda i: (i, 0)),
            pl.BlockSpec((RB, D), lambda i: (i, 0)),
        ],
        out_shape=[
            jax.ShapeDtypeStruct((N, D), jnp.float32),
            jax.ShapeDtypeStruct((N, D), jnp.int32),
            jax.ShapeDtypeStruct((N, D), jnp.int32),
        ],
    )(y, st, gamma.reshape(1, D), beta.reshape(1, D), W_T, W_U, b_T)


def _proju_body(a_ref, xp_ref, wt_ref, wu_ref, bt_ref, x_ref, t_ref, u_ref):
    a = jnp.concatenate([a_ref[0], a_ref[1]], axis=1)
    x = jnp.maximum(a + xp_ref[...], 0.0)
    _proj_common(x, wt_ref, wu_ref, bt_ref, x_ref, t_ref, u_ref)


def _proju(aggr2, xp, W_T, W_U, b_T):
    return pl.pallas_call(
        _proju_body,
        grid=(N // RB,),
        in_specs=[
            pl.BlockSpec((2, RB, DH), lambda i: (0, i, 0)),
            pl.BlockSpec((RB, D), lambda i: (i, 0)),
            pl.BlockSpec((D, 2 * D), lambda i: (0, 0)),
            pl.BlockSpec((D, 2 * D), lambda i: (0, 0)),
            pl.BlockSpec((1, 2 * D), lambda i: (0, 0)),
        ],
        out_specs=[
            pl.BlockSpec((RB, D), lambda i: (i, 0)),
            pl.BlockSpec((RB, D), lambda i: (i, 0)),
            pl.BlockSpec((RB, D), lambda i: (i, 0)),
        ],
        out_shape=[
            jax.ShapeDtypeStruct((N, D), jnp.float32),
            jax.ShapeDtypeStruct((N, D), jnp.int32),
            jax.ShapeDtypeStruct((N, D), jnp.int32),
        ],
    )(aggr2, xp, W_T, W_U, b_T)


def _edge_body(gt_ref, gu_ref, ea_ref, we_ref, m_ref):
    ep = jnp.dot(ea_ref[...], we_ref[...], preferred_element_type=jnp.float32)
    ft, st = _unpack_bf16_pair(gt_ref[...])
    fu, su = _unpack_bf16_pair(gu_ref[...])
    f = ft + fu + ep[:, :D]
    s = st + su + ep[:, D:]
    sig = 1.0 / (1.0 + jnp.exp(-f))
    sp = jnp.maximum(s, 0.0) + jnp.log(1.0 + jnp.exp(-jnp.abs(s)))
    msg = sig * sp
    m_ref[...] = jnp.stack([msg[:, :DH], msg[:, DH:]], axis=0)


def _edge_stage(GT, GU, edge_attr, W_E):
    return pl.pallas_call(
        _edge_body,
        grid=(E // EB,),
        in_specs=[
            pl.BlockSpec((EB, D), lambda i: (i, 0)),
            pl.BlockSpec((EB, D), lambda i: (i, 0)),
            pl.BlockSpec((EB, ED), lambda i: (i, 0)),
            pl.BlockSpec((ED, 2 * D), lambda i: (0, 0)),
        ],
        out_specs=pl.BlockSpec((2, EB, DH), lambda i: (0, i, 0)),
        out_shape=jax.ShapeDtypeStruct((2, E, DH), jnp.float32),
    )(GT, GU, edge_attr, W_E)


def _update_body(a_ref, x_ref, o_ref):
    a = jnp.concatenate([a_ref[0], a_ref[1]], axis=1)
    o_ref[...] = jnp.maximum(a + x_ref[...], 0.0)


def _update(aggr2, x):
    return pl.pallas_call(
        _update_body,
        grid=(N // RB,),
        in_specs=[
            pl.BlockSpec((2, RB, DH), lambda i: (0, i, 0)),
            pl.BlockSpec((RB, D), lambda i: (i, 0)),
        ],
        out_specs=pl.BlockSpec((RB, D), lambda i: (i, 0)),
        out_shape=jax.ShapeDtypeStruct((N, D), jnp.float32),
    )(aggr2, x)


def _gather_body(t_hbm, u_hbm, dst_hbm, src_hbm, gt_hbm, gu_hbm,
                 dbufs, sbufs, tbufs, ubufs, semt, semu, semwt, semwu):
    wid = lax.axis_index("s") * NC + lax.axis_index("c")
    base = wid * G_PER

    def start(j, b, wait_prev):
        off = base + j * GKG
        pltpu.sync_copy(dst_hbm.at[pl.ds(off, GKG)], dbufs[b])
        pltpu.sync_copy(src_hbm.at[pl.ds(off, GKG)], sbufs[b])
        if wait_prev:
            pltpu.make_async_copy(tbufs[b], gt_hbm.at[pl.ds(off, GKG)],
                                  semwt[b]).wait()
            pltpu.make_async_copy(ubufs[b], gu_hbm.at[pl.ds(off, GKG)],
                                  semwu[b]).wait()
        pltpu.async_copy(t_hbm.at[dbufs[b]], tbufs[b], semt[b])
        pltpu.async_copy(u_hbm.at[sbufs[b]], ubufs[b], semu[b])

    def finish(j, b):
        off = base + j * GKG
        pltpu.make_async_copy(t_hbm.at[dbufs[b]], tbufs[b], semt[b]).wait()
        pltpu.make_async_copy(u_hbm.at[sbufs[b]], ubufs[b], semu[b]).wait()
        pltpu.async_copy(tbufs[b], gt_hbm.at[pl.ds(off, GKG)], semwt[b])
        pltpu.async_copy(ubufs[b], gu_hbm.at[pl.ds(off, GKG)], semwu[b])

    start(0, 0, False)
    start(1, 1, False)

    def pair(p, _):
        j0 = 2 * p
        finish(j0, 0)
        start(j0 + 2, 0, True)
        finish(j0 + 1, 1)
        start(j0 + 3, 1, True)
        return 0

    lax.fori_loop(0, G_FULL // 2 - 1, pair, 0)
    finish(G_FULL - 2, 0)
    finish(G_FULL - 1, 1)

    off = base + G_LAST
    pltpu.sync_copy(dst_hbm.at[pl.ds(off, GKG)], dbufs[0])
    pltpu.sync_copy(src_hbm.at[pl.ds(off, GKG)], sbufs[0])
    pltpu.make_async_copy(tbufs[0], gt_hbm.at[pl.ds(off, GKG)], semwt[0]).wait()
    pltpu.make_async_copy(ubufs[0], gu_hbm.at[pl.ds(off, GKG)], semwu[0]).wait()
    pltpu.make_async_copy(tbufs[1], gt_hbm.at[pl.ds(off, GKG)], semwt[1]).wait()
    pltpu.make_async_copy(ubufs[1], gu_hbm.at[pl.ds(off, GKG)], semwu[1]).wait()
    pltpu.async_copy(t_hbm.at[dbufs[0]], tbufs[0], semt[0]).wait()
    pltpu.async_copy(u_hbm.at[sbufs[0]], ubufs[0], semu[0]).wait()
    pltpu.sync_copy(tbufs[0], gt_hbm.at[pl.ds(off, GKG)])
    pltpu.sync_copy(ubufs[0], gu_hbm.at[pl.ds(off, GKG)])


def _gather(T, U, dst, src):
    mesh = plsc.VectorSubcoreMesh(core_axis_name="c", subcore_axis_name="s")
    k = pl.kernel(
        _gather_body,
        out_type=[
            jax.ShapeDtypeStruct((E, D), jnp.int32),
            jax.ShapeDtypeStruct((E, D), jnp.int32),
        ],
        mesh=mesh,
        scratch_types=[
            [pltpu.VMEM((GKG,), jnp.int32)] * 2,
            [pltpu.VMEM((GKG,), jnp.int32)] * 2,
            [pltpu.VMEM((GKG, D), jnp.int32)] * 2,
            [pltpu.VMEM((GKG, D), jnp.int32)] * 2,
            [pltpu.SemaphoreType.DMA] * 2,
            [pltpu.SemaphoreType.DMA] * 2,
            [pltpu.SemaphoreType.DMA] * 2,
            [pltpu.SemaphoreType.DMA] * 2,
        ],
    )
    return k(T, U, dst, src)


def _scatter_body(m_hbm, dst_hbm, a_hbm, spbuf, mbufs, ibufs, mbuf_t, ibuf_t, sems):
    c = lax.axis_index("c")
    sid = lax.axis_index("s")
    mbuf = mbufs[0]

    def zrow(i, _):
        r = i // (DH // 16)
        k = i % (DH // 16)
        mbuf[r, pl.ds(k * 16, 16)] = jnp.zeros((16,), jnp.float32)
        return 0

    lax.fori_loop(0, GK * (DH // 16), zrow, 0)

    start = sid * TROW
    for t in range(TROW // GK):
        pltpu.sync_copy(mbuf, spbuf.at[pl.ds(start + t * GK, GK)])
    pltpu.sync_copy(mbuf.at[pl.ds(0, TROW - (TROW // GK) * GK)],
                    spbuf.at[pl.ds(start + (TROW // GK) * GK,
                                   TROW - (TROW // GK) * GK)])

    @pl.when(sid == NS - 1)
    def _():
        pltpu.sync_copy(mbuf.at[pl.ds(0, N - NS * TROW)],
                        spbuf.at[pl.ds(NS * TROW, N - NS * TROW)])

    plsc.subcore_barrier()

    base = sid * S_PER

    def start(j, b):
        off = base + j * GK
        pltpu.sync_copy(dst_hbm.at[pl.ds(off, GK)], ibufs[b])
        pltpu.async_copy(m_hbm.at[c, pl.ds(off, GK)], mbufs[b], sems[b])

    def finish(j, b):
        off = base + j * GK
        pltpu.make_async_copy(m_hbm.at[c, pl.ds(off, GK)], mbufs[b],
                              sems[b]).wait()
        pltpu.sync_copy(mbufs[b], spbuf.at[ibufs[b]], add=True)

    start(0, 0)
    start(1, 1)

    def pair(p, _):
        j0 = 2 * p
        finish(j0, 0)
        start(j0 + 2, 0)
        finish(j0 + 1, 1)
        start(j0 + 3, 1)
        return 0

    lax.fori_loop(0, S_FULL // 2 - 1, pair, 0)
    finish(S_FULL - 2, 0)
    finish(S_FULL - 1, 1)

    off = base + S_FULL * GK
    pltpu.sync_copy(dst_hbm.at[pl.ds(off, S_TAIL)], ibuf_t)
    pltpu.sync_copy(m_hbm.at[c, pl.ds(off, S_TAIL)], mbuf_t)
    pltpu.sync_copy(mbuf_t, spbuf.at[ibuf_t], add=True)

    plsc.subcore_barrier()
    pltpu.sync_copy(spbuf.at[pl.ds(sid * TROW, TROW)],
                    a_hbm.at[c, pl.ds(sid * TROW, TROW)])

    @pl.when(sid == NS - 1)
    def _():
        pltpu.sync_copy(spbuf.at[pl.ds(NS * TROW, N - NS * TROW)],
                        a_hbm.at[c, pl.ds(NS * TROW, N - NS * TROW)])


def _scatter(msg2, dst):
    mesh = plsc.VectorSubcoreMesh(core_axis_name="c", subcore_axis_name="s")
    k = pl.kernel(
        _scatter_body,
        out_type=jax.ShapeDtypeStruct((2, N, DH), jnp.float32),
        mesh=mesh,
        scratch_types=[
            pltpu.VMEM_SHARED((N, DH), jnp.float32),
            [pltpu.VMEM((GK, DH), jnp.float32)] * 2,
            [pltpu.VMEM((GK,), jnp.int32)] * 2,
            pltpu.VMEM((S_TAIL, DH), jnp.float32),
            pltpu.VMEM((S_TAIL,), jnp.int32),
            [pltpu.SemaphoreType.DMA] * 2,
        ],
    )
    return k(msg2, dst)


def kernel(node_attr, edge_index, edge_attr, W0, b0, gamma, beta, Wf, bf, Ws, bs):
    src = edge_index[0]
    dst = edge_index[1]

    y, st = _mm_stats(node_attr, W0, b0)

    L = Wf.shape[0]
    x = None
    aggr2 = None
    for l in range(L):
        W_T = jnp.concatenate([Wf[l, :D], Ws[l, :D]], axis=1)
        W_U = jnp.concatenate([Wf[l, D:2 * D], Ws[l, D:2 * D]], axis=1)
        W_E = jnp.concatenate([Wf[l, 2 * D:], Ws[l, 2 * D:]], axis=1)
        b_T = jnp.concatenate([bf[l], bs[l]]).reshape(1, 2 * D)
        if l == 0:
            x, T, U = _proj0(y, st, gamma, beta, W_T, W_U, b_T)
        else:
            x, T, U = _proju(aggr2, x, W_T, W_U, b_T)
        GT, GU = _gather(T, U, dst, src)
        msg2 = _edge_stage(GT, GU, edge_attr, W_E)
        aggr2 = _scatter(msg2, dst)
    return _update(aggr2, x)


# speedup vs baseline: 5.4622x; 1.0283x over previous
import jax
import jax.numpy as jnp
from jax import lax
from jax.experimental import pallas as pl
from jax.experimental.pallas import tpu as pltpu
from jax.experimental.pallas import tpu_sc as plsc

N = 10000
E = 160000
D = 256
ED = 16

RB = 2000
EB = 800

NC = 2
NS = 16
NW = NC * NS

GK = 128
GKG = 112
EH = E // 2
G_PER = EH // NW // 8 * 8
G_FULL = G_PER // GKG
G_LAST = G_PER - GKG
G_REM0 = NW * G_PER
G_REM1 = EH - GKG

S_PER = E // NS
S_FULL = S_PER // GK
S_TAIL = S_PER - S_FULL * GK

DH = D // 2
TROW = 624


def _mm_stats_body(x_ref, w_ref, b_ref, y_ref, st_ref):
    i = pl.program_id(0)
    y = jnp.dot(x_ref[...], w_ref[...], preferred_element_type=jnp.float32)
    y = y + b_ref[...]
    y_ref[...] = y
    s1 = jnp.sum(y, axis=0)
    s2 = jnp.sum(y * y, axis=0)
    rows = lax.broadcasted_iota(jnp.int32, (8, D), 0)
    upd = jnp.where(rows == 0, s1[None, :], 0.0) + jnp.where(rows == 1, s2[None, :], 0.0)

    @pl.when(i == 0)
    def _():
        st_ref[...] = jnp.zeros_like(st_ref)

    st_ref[...] += upd


def _mm_stats(node_attr, W0, b0):
    return pl.pallas_call(
        _mm_stats_body,
        grid=(N // RB,),
        in_specs=[
            pl.BlockSpec((RB, D), lambda i: (i, 0)),
            pl.BlockSpec((D, D), lambda i: (0, 0)),
            pl.BlockSpec((1, D), lambda i: (0, 0)),
        ],
        out_specs=[
            pl.BlockSpec((RB, D), lambda i: (i, 0)),
            pl.BlockSpec((8, D), lambda i: (0, 0)),
        ],
        out_shape=[
            jax.ShapeDtypeStruct((N, D), jnp.float32),
            jax.ShapeDtypeStruct((8, D), jnp.float32),
        ],
    )(node_attr, W0, b0.reshape(1, D))


def _pack_bf16_pair(lo, hi):
    ul = lax.bitcast_convert_type(lo, jnp.int32)
    uh = lax.bitcast_convert_type(hi, jnp.int32)
    ul = ul + 0x7FFF + ((ul >> 16) & 1)
    uh = uh + 0x7FFF + ((uh >> 16) & 1)
    return ((ul >> 16) & 0xFFFF) | (uh & jnp.int32(-65536))


def _unpack_bf16_pair(p):
    lo = lax.bitcast_convert_type(p << 16, jnp.float32)
    hi = lax.bitcast_convert_type(p & jnp.int32(-65536), jnp.float32)
    return lo, hi


def _proj_common(x, wt_ref, wu_ref, bt_ref, x_ref, t_ref, u_ref):
    x_ref[...] = x
    t = jnp.dot(x, wt_ref[...], preferred_element_type=jnp.float32) + bt_ref[...]
    u = jnp.dot(x, wu_ref[...], preferred_element_type=jnp.float32)
    t_ref[...] = _pack_bf16_pair(t[:, :D], t[:, D:])
    u_ref[...] = _pack_bf16_pair(u[:, :D], u[:, D:])


def _proj0_body(y_ref, st_ref, g_ref, be_ref, wt_ref, wu_ref, bt_ref,
                x_ref, t_ref, u_ref):
    mean = st_ref[0:1, :] / N
    var = st_ref[1:2, :] / N - mean * mean
    inv = g_ref[...] * lax.rsqrt(var + 1e-5)
    x = jnp.maximum((y_ref[...] - mean) * inv + be_ref[...], 0.0)
    _proj_common(x, wt_ref, wu_ref, bt_ref, x_ref, t_ref, u_ref)


def _proj0(y, st, gamma, beta, W_T, W_U, b_T):
    return pl.pallas_call(
        _proj0_body,
        grid=(N // RB,),
        in_specs=[
            pl.BlockSpec((RB, D), lambda i: (i, 0)),
            pl.BlockSpec((8, D), lambda i: (0, 0)),
            pl.BlockSpec((1, D), lambda i: (0, 0)),
            pl.BlockSpec((1, D), lambda i: (0, 0)),
            pl.BlockSpec((D, 2 * D), lambda i: (0, 0)),
            pl.BlockSpec((D, 2 * D), lambda i: (0, 0)),
            pl.BlockSpec((1, 2 * D), lambda i: (0, 0)),
        ],
        out_specs=[
            pl.BlockSpec((RB, D), lambda i: (i, 0)),
            pl.BlockSpec((RB, D), lambda i: (i, 0)),
            pl.BlockSpec((RB, D), lambda i: (i, 0)),
        ],
        out_shape=[
            jax.ShapeDtypeStruct((N, D), jnp.float32),
            jax.ShapeDtypeStruct((N, D), jnp.int32),
            jax.ShapeDtypeStruct((N, D), jnp.int32),
        ],
    )(y, st, gamma.reshape(1, D), beta.reshape(1, D), W_T, W_U, b_T)


def _proju_body(a_ref, xp_ref, wt_ref, wu_ref, bt_ref, x_ref, t_ref, u_ref):
    a = jnp.concatenate([a_ref[0], a_ref[1]], axis=1)
    x = jnp.maximum(a + xp_ref[...], 0.0)
    _proj_common(x, wt_ref, wu_ref, bt_ref, x_ref, t_ref, u_ref)


def _proju(aggr2, xp, W_T, W_U, b_T):
    return pl.pallas_call(
        _proju_body,
        grid=(N // RB,),
        in_specs=[
            pl.BlockSpec((2, RB, DH), lambda i: (0, i, 0)),
            pl.BlockSpec((RB, D), lambda i: (i, 0)),
            pl.BlockSpec((D, 2 * D), lambda i: (0, 0)),
            pl.BlockSpec((D, 2 * D), lambda i: (0, 0)),
            pl.BlockSpec((1, 2 * D), lambda i: (0, 0)),
        ],
        out_specs=[
            pl.BlockSpec((RB, D), lambda i: (i, 0)),
            pl.BlockSpec((RB, D), lambda i: (i, 0)),
            pl.BlockSpec((RB, D), lambda i: (i, 0)),
        ],
        out_shape=[
            jax.ShapeDtypeStruct((N, D), jnp.float32),
            jax.ShapeDtypeStruct((N, D), jnp.int32),
            jax.ShapeDtypeStruct((N, D), jnp.int32),
        ],
    )(aggr2, xp, W_T, W_U, b_T)


def _edge_body(gt_ref, gu_ref, ea_ref, we_ref, m_ref):
    ep = jnp.dot(ea_ref[...], we_ref[...], preferred_element_type=jnp.float32)
    ft, st = _unpack_bf16_pair(gt_ref[...])
    fu, su = _unpack_bf16_pair(gu_ref[...])
    f = ft + fu + ep[:, :D]
    s = st + su + ep[:, D:]
    sig = 1.0 / (1.0 + jnp.exp(-f))
    sp = jnp.maximum(s, 0.0) + jnp.log(1.0 + jnp.exp(-jnp.abs(s)))
    msg = sig * sp
    m_ref[...] = jnp.stack([msg[:, :DH], msg[:, DH:]], axis=0)


def _edge_stage(GT, GU, edge_attr, W_E, part):
    off = part * (EH // EB)
    return pl.pallas_call(
        _edge_body,
        grid=(EH // EB,),
        in_specs=[
            pl.BlockSpec((EB, D), lambda i: (i, 0)),
            pl.BlockSpec((EB, D), lambda i: (i, 0)),
            pl.BlockSpec((EB, ED), lambda i: (i + off, 0)),
            pl.BlockSpec((ED, 2 * D), lambda i: (0, 0)),
        ],
        out_specs=pl.BlockSpec((2, EB, DH), lambda i: (0, i, 0)),
        out_shape=jax.ShapeDtypeStruct((2, EH, DH), jnp.float32),
    )(GT, GU, edge_attr, W_E)


def _update_body(a_ref, x_ref, o_ref):
    a = jnp.concatenate([a_ref[0], a_ref[1]], axis=1)
    o_ref[...] = jnp.maximum(a + x_ref[...], 0.0)


def _update(aggr2, x):
    return pl.pallas_call(
        _update_body,
        grid=(N // RB,),
        in_specs=[
            pl.BlockSpec((2, RB, DH), lambda i: (0, i, 0)),
            pl.BlockSpec((RB, D), lambda i: (i, 0)),
        ],
        out_specs=pl.BlockSpec((RB, D), lambda i: (i, 0)),
        out_shape=jax.ShapeDtypeStruct((N, D), jnp.float32),
    )(aggr2, x)


def _make_gather_body(part):
    def body(t_hbm, u_hbm, dst_hbm, src_hbm, gt_hbm, gu_hbm,
             dbufs, sbufs, tbufs, ubufs, semt, semu, semwt, semwu):
        wid = lax.axis_index("s") * NC + lax.axis_index("c")
        base = wid * G_PER
        ibase = part * EH + base

        def start(j, b, wait_prev):
            off = base + j * GKG
            pltpu.sync_copy(dst_hbm.at[pl.ds(ibase + j * GKG, GKG)], dbufs[b])
            pltpu.sync_copy(src_hbm.at[pl.ds(ibase + j * GKG, GKG)], sbufs[b])
            if wait_prev:
                pltpu.make_async_copy(tbufs[b], gt_hbm.at[pl.ds(off, GKG)],
                                      semwt[b]).wait()
                pltpu.make_async_copy(ubufs[b], gu_hbm.at[pl.ds(off, GKG)],
                                      semwu[b]).wait()
            pltpu.async_copy(t_hbm.at[dbufs[b]], tbufs[b], semt[b])
            pltpu.async_copy(u_hbm.at[sbufs[b]], ubufs[b], semu[b])

        def finish(j, b):
            off = base + j * GKG
            pltpu.make_async_copy(t_hbm.at[dbufs[b]], tbufs[b], semt[b]).wait()
            pltpu.make_async_copy(u_hbm.at[sbufs[b]], ubufs[b], semu[b]).wait()
            pltpu.async_copy(tbufs[b], gt_hbm.at[pl.ds(off, GKG)], semwt[b])
            pltpu.async_copy(ubufs[b], gu_hbm.at[pl.ds(off, GKG)], semwu[b])

        start(0, 0, False)
        start(1, 1, False)

        def pair(p, _):
            j0 = 2 * p
            finish(j0, 0)
            start(j0 + 2, 0, True)
            finish(j0 + 1, 1)
            start(j0 + 3, 1, True)
            return 0

        lax.fori_loop(0, G_FULL // 2 - 1, pair, 0)
        finish(G_FULL - 2, 0)
        finish(G_FULL - 1, 1)

        off0 = base + G_LAST
        pltpu.make_async_copy(tbufs[0], gt_hbm.at[pl.ds(off0, GKG)], semwt[0]).wait()
        pltpu.make_async_copy(ubufs[0], gu_hbm.at[pl.ds(off0, GKG)], semwu[0]).wait()
        pltpu.make_async_copy(tbufs[1], gt_hbm.at[pl.ds(off0, GKG)], semwt[1]).wait()
        pltpu.make_async_copy(ubufs[1], gu_hbm.at[pl.ds(off0, GKG)], semwu[1]).wait()

        def sync_chunk(o):
            pltpu.sync_copy(dst_hbm.at[pl.ds(part * EH + o, GKG)], dbufs[0])
            pltpu.sync_copy(src_hbm.at[pl.ds(part * EH + o, GKG)], sbufs[0])
            pltpu.async_copy(t_hbm.at[dbufs[0]], tbufs[0], semt[0]).wait()
            pltpu.async_copy(u_hbm.at[sbufs[0]], ubufs[0], semu[0]).wait()
            pltpu.sync_copy(tbufs[0], gt_hbm.at[pl.ds(o, GKG)])
            pltpu.sync_copy(ubufs[0], gu_hbm.at[pl.ds(o, GKG)])

        sync_chunk(base + G_LAST)

        @pl.when(wid == 0)
        def _():
            sync_chunk(G_REM0)
            sync_chunk(G_REM1)

    return body


def _gather(T, U, dst, src, part):
    mesh = plsc.VectorSubcoreMesh(core_axis_name="c", subcore_axis_name="s")
    k = pl.kernel(
        _make_gather_body(part),
        out_type=[
            jax.ShapeDtypeStruct((EH, D), jnp.int32),
            jax.ShapeDtypeStruct((EH, D), jnp.int32),
        ],
        mesh=mesh,
        scratch_types=[
            [pltpu.VMEM((GKG,), jnp.int32)] * 2,
            [pltpu.VMEM((GKG,), jnp.int32)] * 2,
            [pltpu.VMEM((GKG, D), jnp.int32)] * 2,
            [pltpu.VMEM((GKG, D), jnp.int32)] * 2,
            [pltpu.SemaphoreType.DMA] * 2,
            [pltpu.SemaphoreType.DMA] * 2,
            [pltpu.SemaphoreType.DMA] * 2,
            [pltpu.SemaphoreType.DMA] * 2,
        ],
    )
    return k(T, U, dst, src)


def _scatter_body(m0_hbm, m1_hbm, dst_hbm, a_hbm,
                  spbuf, mbufs, ibufs, mbuf_t, ibuf_t, sems):
    c = lax.axis_index("c")
    sid = lax.axis_index("s")
    mbuf = mbufs[0]

    def zrow(i, _):
        r = i // (DH // 16)
        k = i % (DH // 16)
        mbuf[r, pl.ds(k * 16, 16)] = jnp.zeros((16,), jnp.float32)
        return 0

    lax.fori_loop(0, GK * (DH // 16), zrow, 0)

    start = sid * TROW
    for t in range(TROW // GK):
        pltpu.sync_copy(mbuf, spbuf.at[pl.ds(start + t * GK, GK)])
    pltpu.sync_copy(mbuf.at[pl.ds(0, TROW - (TROW // GK) * GK)],
                    spbuf.at[pl.ds(start + (TROW // GK) * GK,
                                   TROW - (TROW // GK) * GK)])

    @pl.when(sid == NS - 1)
    def _():
        pltpu.sync_copy(mbuf.at[pl.ds(0, N - NS * TROW)],
                        spbuf.at[pl.ds(NS * TROW, N - NS * TROW)])

    plsc.subcore_barrier()

    base = sid * S_PER

    def run_part(m_hbm, lbase):
        def start(j, b):
            pltpu.sync_copy(dst_hbm.at[pl.ds(base + j * GK, GK)], ibufs[b])
            pltpu.async_copy(m_hbm.at[c, pl.ds(lbase + j * GK, GK)],
                             mbufs[b], sems[b])

        def finish(j, b):
            pltpu.make_async_copy(m_hbm.at[c, pl.ds(lbase + j * GK, GK)],
                                  mbufs[b], sems[b]).wait()
            pltpu.sync_copy(mbufs[b], spbuf.at[ibufs[b]], add=True)

        start(0, 0)
        start(1, 1)

        def pair(p, _):
            j0 = 2 * p
            finish(j0, 0)
            start(j0 + 2, 0)
            finish(j0 + 1, 1)
            start(j0 + 3, 1)
            return 0

        lax.fori_loop(0, S_FULL // 2 - 1, pair, 0)
        finish(S_FULL - 2, 0)
        finish(S_FULL - 1, 1)

        pltpu.sync_copy(dst_hbm.at[pl.ds(base + S_FULL * GK, S_TAIL)], ibuf_t)
        pltpu.sync_copy(m_hbm.at[c, pl.ds(lbase + S_FULL * GK, S_TAIL)], mbuf_t)
        pltpu.sync_copy(mbuf_t, spbuf.at[ibuf_t], add=True)

    @pl.when(sid < NS // 2)
    def _():
        run_part(m0_hbm, sid * S_PER)

    @pl.when(sid >= NS // 2)
    def _():
        run_part(m1_hbm, sid * S_PER - EH)

    plsc.subcore_barrier()
    pltpu.sync_copy(spbuf.at[pl.ds(sid * TROW, TROW)],
                    a_hbm.at[c, pl.ds(sid * TROW, TROW)])

    @pl.when(sid == NS - 1)
    def _():
        pltpu.sync_copy(spbuf.at[pl.ds(NS * TROW, N - NS * TROW)],
                        a_hbm.at[c, pl.ds(NS * TROW, N - NS * TROW)])


def _scatter(m0, m1, dst):
    mesh = plsc.VectorSubcoreMesh(core_axis_name="c", subcore_axis_name="s")
    k = pl.kernel(
        _scatter_body,
        out_type=jax.ShapeDtypeStruct((2, N, DH), jnp.float32),
        mesh=mesh,
        scratch_types=[
            pltpu.VMEM_SHARED((N, DH), jnp.float32),
            [pltpu.VMEM((GK, DH), jnp.float32)] * 2,
            [pltpu.VMEM((GK,), jnp.int32)] * 2,
            pltpu.VMEM((S_TAIL, DH), jnp.float32),
            pltpu.VMEM((S_TAIL,), jnp.int32),
            [pltpu.SemaphoreType.DMA] * 2,
        ],
    )
    return k(m0, m1, dst)


def kernel(node_attr, edge_index, edge_attr, W0, b0, gamma, beta, Wf, bf, Ws, bs):
    src = edge_index[0]
    dst = edge_index[1]

    y, st = _mm_stats(node_attr, W0, b0)

    L = Wf.shape[0]
    x = None
    aggr2 = None
    for l in range(L):
        W_T = jnp.concatenate([Wf[l, :D], Ws[l, :D]], axis=1)
        W_U = jnp.concatenate([Wf[l, D:2 * D], Ws[l, D:2 * D]], axis=1)
        W_E = jnp.concatenate([Wf[l, 2 * D:], Ws[l, 2 * D:]], axis=1)
        b_T = jnp.concatenate([bf[l], bs[l]]).reshape(1, 2 * D)
        if l == 0:
            x, T, U = _proj0(y, st, gamma, beta, W_T, W_U, b_T)
        else:
            x, T, U = _proju(aggr2, x, W_T, W_U, b_T)
        GT0, GU0 = _gather(T, U, dst, src, 0)
        m0 = _edge_stage(GT0, GU0, edge_attr, W_E, 0)
        GT1, GU1 = _gather(T, U, dst, src, 1)
        m1 = _edge_stage(GT1, GU1, edge_attr, W_E, 1)
        aggr2 = _scatter(m0, m1, dst)
    return _update(aggr2, x)


# speedup vs baseline: 5.5975x; 1.0248x over previous
import jax
import jax.numpy as jnp
from jax import lax
from jax.experimental import pallas as pl
from jax.experimental.pallas import tpu as pltpu
from jax.experimental.pallas import tpu_sc as plsc

N = 10000
E = 160000
D = 256
ED = 16

RB = 2000
EB = 800

NC = 2
NS = 16
NW = NC * NS

GK = 128
GKG = 112
EH = E // 2
G_PER = EH // NW // 8 * 8
G_FULL = G_PER // GKG
G_LAST = G_PER - GKG
G_REM0 = NW * G_PER
G_REM1 = EH - GKG

S_PER = E // NS
S_FULL = S_PER // GK
S_TAIL = S_PER - S_FULL * GK

DH = D // 2
TROW = 624


def _mm_stats_body(x_ref, w_ref, b_ref, y_ref, st_ref):
    i = pl.program_id(0)
    y = jnp.dot(x_ref[...], w_ref[...], preferred_element_type=jnp.float32)
    y = y + b_ref[...]
    y_ref[...] = y
    s1 = jnp.sum(y, axis=0)
    s2 = jnp.sum(y * y, axis=0)
    rows = lax.broadcasted_iota(jnp.int32, (8, D), 0)
    upd = jnp.where(rows == 0, s1[None, :], 0.0) + jnp.where(rows == 1, s2[None, :], 0.0)

    @pl.when(i == 0)
    def _():
        st_ref[...] = jnp.zeros_like(st_ref)

    st_ref[...] += upd


def _mm_stats(node_attr, W0, b0):
    return pl.pallas_call(
        _mm_stats_body,
        grid=(N // RB,),
        in_specs=[
            pl.BlockSpec((RB, D), lambda i: (i, 0)),
            pl.BlockSpec((D, D), lambda i: (0, 0)),
            pl.BlockSpec((1, D), lambda i: (0, 0)),
        ],
        out_specs=[
            pl.BlockSpec((RB, D), lambda i: (i, 0)),
            pl.BlockSpec((8, D), lambda i: (0, 0)),
        ],
        out_shape=[
            jax.ShapeDtypeStruct((N, D), jnp.float32),
            jax.ShapeDtypeStruct((8, D), jnp.float32),
        ],
    )(node_attr, W0, b0.reshape(1, D))


def _pack_bf16_pair(lo, hi):
    ul = lax.bitcast_convert_type(lo, jnp.int32)
    uh = lax.bitcast_convert_type(hi, jnp.int32)
    ul = ul + 0x7FFF + ((ul >> 16) & 1)
    uh = uh + 0x7FFF + ((uh >> 16) & 1)
    return ((ul >> 16) & 0xFFFF) | (uh & jnp.int32(-65536))


def _unpack_bf16_pair(p):
    lo = lax.bitcast_convert_type(p << 16, jnp.float32)
    hi = lax.bitcast_convert_type(p & jnp.int32(-65536), jnp.float32)
    return lo, hi


def _proj_common(x, wt_ref, wu_ref, bt_ref, x_ref, t_ref, u_ref):
    x_ref[...] = x
    t = jnp.dot(x, wt_ref[...], preferred_element_type=jnp.float32) + bt_ref[...]
    u = jnp.dot(x, wu_ref[...], preferred_element_type=jnp.float32)
    t_ref[...] = _pack_bf16_pair(t[:, :D], t[:, D:])
    u_ref[...] = _pack_bf16_pair(u[:, :D], u[:, D:])


def _proj0_body(y_ref, st_ref, g_ref, be_ref, wt_ref, wu_ref, bt_ref,
                x_ref, t_ref, u_ref):
    mean = st_ref[0:1, :] / N
    var = st_ref[1:2, :] / N - mean * mean
    inv = g_ref[...] * lax.rsqrt(var + 1e-5)
    x = jnp.maximum((y_ref[...] - mean) * inv + be_ref[...], 0.0)
    _proj_common(x, wt_ref, wu_ref, bt_ref, x_ref, t_ref, u_ref)


def _proj0(y, st, gamma, beta, W_T, W_U, b_T):
    return pl.pallas_call(
        _proj0_body,
        grid=(N // RB,),
        in_specs=[
            pl.BlockSpec((RB, D), lambda i: (i, 0)),
            pl.BlockSpec((8, D), lambda i: (0, 0)),
            pl.BlockSpec((1, D), lambda i: (0, 0)),
            pl.BlockSpec((1, D), lambda i: (0, 0)),
            pl.BlockSpec((D, 2 * D), lambda i: (0, 0)),
            pl.BlockSpec((D, 2 * D), lambda i: (0, 0)),
            pl.BlockSpec((1, 2 * D), lambda i: (0, 0)),
        ],
        out_specs=[
            pl.BlockSpec((RB, D), lambda i: (i, 0)),
            pl.BlockSpec((RB, D), lambda i: (i, 0)),
            pl.BlockSpec((RB, D), lambda i: (i, 0)),
        ],
        out_shape=[
            jax.ShapeDtypeStruct((N, D), jnp.float32),
            jax.ShapeDtypeStruct((N, D), jnp.int32),
            jax.ShapeDtypeStruct((N, D), jnp.int32),
        ],
    )(y, st, gamma.reshape(1, D), beta.reshape(1, D), W_T, W_U, b_T)


def _proju_body(a_ref, xp_ref, wt_ref, wu_ref, bt_ref, x_ref, t_ref, u_ref):
    a = jnp.concatenate([a_ref[0], a_ref[1]], axis=1)
    x = jnp.maximum(a + xp_ref[...], 0.0)
    _proj_common(x, wt_ref, wu_ref, bt_ref, x_ref, t_ref, u_ref)


def _proju(aggr2, xp, W_T, W_U, b_T):
    return pl.pallas_call(
        _proju_body,
        grid=(N // RB,),
        in_specs=[
            pl.BlockSpec((2, RB, DH), lambda i: (0, i, 0)),
            pl.BlockSpec((RB, D), lambda i: (i, 0)),
            pl.BlockSpec((D, 2 * D), lambda i: (0, 0)),
            pl.BlockSpec((D, 2 * D), lambda i: (0, 0)),
            pl.BlockSpec((1, 2 * D), lambda i: (0, 0)),
        ],
        out_specs=[
            pl.BlockSpec((RB, D), lambda i: (i, 0)),
            pl.BlockSpec((RB, D), lambda i: (i, 0)),
            pl.BlockSpec((RB, D), lambda i: (i, 0)),
        ],
        out_shape=[
            jax.ShapeDtypeStruct((N, D), jnp.float32),
            jax.ShapeDtypeStruct((N, D), jnp.int32),
            jax.ShapeDtypeStruct((N, D), jnp.int32),
        ],
    )(aggr2, xp, W_T, W_U, b_T)


def _edge_body(gt_ref, gu_ref, ea_ref, we_ref, m_ref):
    ep = jnp.dot(ea_ref[...], we_ref[...], preferred_element_type=jnp.float32)
    ft, st = _unpack_bf16_pair(gt_ref[...])
    fu, su = _unpack_bf16_pair(gu_ref[...])
    f = ft + fu + ep[:, :D]
    s = st + su + ep[:, D:]
    sig = 1.0 / (1.0 + jnp.exp(-f))
    sp = jnp.maximum(s, 0.0) + jnp.log(1.0 + jnp.exp(-jnp.abs(s)))
    msg = sig * sp
    m_ref[...] = jnp.stack([msg[:, :DH], msg[:, DH:]], axis=0)


def _edge_stage(GT, GU, edge_attr, W_E, part):
    off = part * (EH // EB)
    return pl.pallas_call(
        _edge_body,
        grid=(EH // EB,),
        in_specs=[
            pl.BlockSpec((EB, D), lambda i: (i, 0)),
            pl.BlockSpec((EB, D), lambda i: (i, 0)),
            pl.BlockSpec((EB, ED), lambda i: (i + off, 0)),
            pl.BlockSpec((ED, 2 * D), lambda i: (0, 0)),
        ],
        out_specs=pl.BlockSpec((2, EB, DH), lambda i: (0, i, 0)),
        out_shape=jax.ShapeDtypeStruct((2, EH, DH), jnp.float32),
    )(GT, GU, edge_attr, W_E)


def _update_body(a_ref, x_ref, o_ref):
    a = jnp.concatenate([a_ref[0], a_ref[1]], axis=1)
    o_ref[...] = jnp.maximum(a + x_ref[...], 0.0)


def _update(aggr2, x):
    return pl.pallas_call(
        _update_body,
        grid=(N // RB,),
        in_specs=[
            pl.BlockSpec((2, RB, DH), lambda i: (0, i, 0)),
            pl.BlockSpec((RB, D), lambda i: (i, 0)),
        ],
        out_specs=pl.BlockSpec((RB, D), lambda i: (i, 0)),
        out_shape=jax.ShapeDtypeStruct((N, D), jnp.float32),
    )(aggr2, x)


def _make_gather_body(part):
    def body(t_hbm, u_hbm, dst_hbm, src_hbm, gt_hbm, gu_hbm,
             dbufs, sbufs, tbufs, ubufs, semt, semu, semwt, semwu):
        wid = lax.axis_index("s") * NC + lax.axis_index("c")
        base = wid * G_PER
        ibase = part * EH + base

        def start(j, b, wait_prev):
            off = base + j * GKG
            pltpu.sync_copy(dst_hbm.at[pl.ds(ibase + j * GKG, GKG)], dbufs[b])
            pltpu.sync_copy(src_hbm.at[pl.ds(ibase + j * GKG, GKG)], sbufs[b])
            if wait_prev:
                pltpu.make_async_copy(tbufs[b], gt_hbm.at[pl.ds(off, GKG)],
                                      semwt[b]).wait()
                pltpu.make_async_copy(ubufs[b], gu_hbm.at[pl.ds(off, GKG)],
                                      semwu[b]).wait()
            pltpu.async_copy(t_hbm.at[dbufs[b]], tbufs[b], semt[b])
            pltpu.async_copy(u_hbm.at[sbufs[b]], ubufs[b], semu[b])

        def finish(j, b):
            off = base + j * GKG
            pltpu.make_async_copy(t_hbm.at[dbufs[b]], tbufs[b], semt[b]).wait()
            pltpu.make_async_copy(u_hbm.at[sbufs[b]], ubufs[b], semu[b]).wait()
            pltpu.async_copy(tbufs[b], gt_hbm.at[pl.ds(off, GKG)], semwt[b])
            pltpu.async_copy(ubufs[b], gu_hbm.at[pl.ds(off, GKG)], semwu[b])

        start(0, 0, False)
        start(1, 1, False)

        def pair(p, _):
            j0 = 2 * p
            finish(j0, 0)
            start(j0 + 2, 0, True)
            finish(j0 + 1, 1)
            start(j0 + 3, 1, True)
            return 0

        lax.fori_loop(0, G_FULL // 2 - 1, pair, 0)
        finish(G_FULL - 2, 0)
        finish(G_FULL - 1, 1)

        off0 = base + G_LAST
        pltpu.make_async_copy(tbufs[0], gt_hbm.at[pl.ds(off0, GKG)], semwt[0]).wait()
        pltpu.make_async_copy(ubufs[0], gu_hbm.at[pl.ds(off0, GKG)], semwu[0]).wait()
        pltpu.make_async_copy(tbufs[1], gt_hbm.at[pl.ds(off0, GKG)], semwt[1]).wait()
        pltpu.make_async_copy(ubufs[1], gu_hbm.at[pl.ds(off0, GKG)], semwu[1]).wait()

        def sync_chunk(o):
            pltpu.sync_copy(dst_hbm.at[pl.ds(part * EH + o, GKG)], dbufs[0])
            pltpu.sync_copy(src_hbm.at[pl.ds(part * EH + o, GKG)], sbufs[0])
            pltpu.async_copy(t_hbm.at[dbufs[0]], tbufs[0], semt[0]).wait()
            pltpu.async_copy(u_hbm.at[sbufs[0]], ubufs[0], semu[0]).wait()
            pltpu.sync_copy(tbufs[0], gt_hbm.at[pl.ds(o, GKG)])
            pltpu.sync_copy(ubufs[0], gu_hbm.at[pl.ds(o, GKG)])

        sync_chunk(base + G_LAST)

        @pl.when(wid == 0)
        def _():
            sync_chunk(G_REM0)
            sync_chunk(G_REM1)

    return body


def _gather(T, U, dst, src, part):
    mesh = plsc.VectorSubcoreMesh(core_axis_name="c", subcore_axis_name="s")
    k = pl.kernel(
        _make_gather_body(part),
        out_type=[
            jax.ShapeDtypeStruct((EH, D), jnp.int32),
            jax.ShapeDtypeStruct((EH, D), jnp.int32),
        ],
        mesh=mesh,
        scratch_types=[
            [pltpu.VMEM((GKG,), jnp.int32)] * 2,
            [pltpu.VMEM((GKG,), jnp.int32)] * 2,
            [pltpu.VMEM((GKG, D), jnp.int32)] * 2,
            [pltpu.VMEM((GKG, D), jnp.int32)] * 2,
            [pltpu.SemaphoreType.DMA] * 2,
            [pltpu.SemaphoreType.DMA] * 2,
            [pltpu.SemaphoreType.DMA] * 2,
            [pltpu.SemaphoreType.DMA] * 2,
        ],
    )
    return k(T, U, dst, src)


def _scatter_body(m0_hbm, m1_hbm, dst_hbm, a_hbm,
                  spbuf, mbufs, ibufs, mbuf_t, ibuf_t, sems, semi):
    c = lax.axis_index("c")
    sid = lax.axis_index("s")
    mbuf = mbufs[0]

    def zrow(i, _):
        r = i // (DH // 16)
        k = i % (DH // 16)
        mbuf[r, pl.ds(k * 16, 16)] = jnp.zeros((16,), jnp.float32)
        return 0

    lax.fori_loop(0, GK * (DH // 16), zrow, 0)

    start = sid * TROW
    for t in range(TROW // GK):
        pltpu.sync_copy(mbuf, spbuf.at[pl.ds(start + t * GK, GK)])
    pltpu.sync_copy(mbuf.at[pl.ds(0, TROW - (TROW // GK) * GK)],
                    spbuf.at[pl.ds(start + (TROW // GK) * GK,
                                   TROW - (TROW // GK) * GK)])

    @pl.when(sid == NS - 1)
    def _():
        pltpu.sync_copy(mbuf.at[pl.ds(0, N - NS * TROW)],
                        spbuf.at[pl.ds(NS * TROW, N - NS * TROW)])

    plsc.subcore_barrier()

    base = sid * S_PER

    def run_part(m_hbm, lbase):
        def start(j, b):
            pltpu.async_copy(dst_hbm.at[pl.ds(base + j * GK, GK)], ibufs[b],
                             semi[b])
            pltpu.async_copy(m_hbm.at[c, pl.ds(lbase + j * GK, GK)],
                             mbufs[b], sems[b])

        def finish(j, b):
            pltpu.make_async_copy(dst_hbm.at[pl.ds(base + j * GK, GK)],
                                  ibufs[b], semi[b]).wait()
            pltpu.make_async_copy(m_hbm.at[c, pl.ds(lbase + j * GK, GK)],
                                  mbufs[b], sems[b]).wait()
            pltpu.sync_copy(mbufs[b], spbuf.at[ibufs[b]], add=True)

        start(0, 0)
        start(1, 1)

        def pair(p, _):
            j0 = 2 * p
            finish(j0, 0)
            start(j0 + 2, 0)
            finish(j0 + 1, 1)
            start(j0 + 3, 1)
            return 0

        lax.fori_loop(0, S_FULL // 2 - 1, pair, 0)
        finish(S_FULL - 2, 0)
        finish(S_FULL - 1, 1)

        pltpu.sync_copy(dst_hbm.at[pl.ds(base + S_FULL * GK, S_TAIL)], ibuf_t)
        pltpu.sync_copy(m_hbm.at[c, pl.ds(lbase + S_FULL * GK, S_TAIL)], mbuf_t)
        pltpu.sync_copy(mbuf_t, spbuf.at[ibuf_t], add=True)

    @pl.when(sid < NS // 2)
    def _():
        run_part(m0_hbm, sid * S_PER)

    @pl.when(sid >= NS // 2)
    def _():
        run_part(m1_hbm, sid * S_PER - EH)

    plsc.subcore_barrier()
    pltpu.sync_copy(spbuf.at[pl.ds(sid * TROW, TROW)],
                    a_hbm.at[c, pl.ds(sid * TROW, TROW)])

    @pl.when(sid == NS - 1)
    def _():
        pltpu.sync_copy(spbuf.at[pl.ds(NS * TROW, N - NS * TROW)],
                        a_hbm.at[c, pl.ds(NS * TROW, N - NS * TROW)])


def _scatter(m0, m1, dst):
    mesh = plsc.VectorSubcoreMesh(core_axis_name="c", subcore_axis_name="s")
    k = pl.kernel(
        _scatter_body,
        out_type=jax.ShapeDtypeStruct((2, N, DH), jnp.float32),
        mesh=mesh,
        scratch_types=[
            pltpu.VMEM_SHARED((N, DH), jnp.float32),
            [pltpu.VMEM((GK, DH), jnp.float32)] * 2,
            [pltpu.VMEM((GK,), jnp.int32)] * 2,
            pltpu.VMEM((S_TAIL, DH), jnp.float32),
            pltpu.VMEM((S_TAIL,), jnp.int32),
            [pltpu.SemaphoreType.DMA] * 2,
            [pltpu.SemaphoreType.DMA] * 2,
        ],
    )
    return k(m0, m1, dst)


def kernel(node_attr, edge_index, edge_attr, W0, b0, gamma, beta, Wf, bf, Ws, bs):
    src = edge_index[0]
    dst = edge_index[1]

    y, st = _mm_stats(node_attr, W0, b0)

    L = Wf.shape[0]
    x = None
    aggr2 = None
    for l in range(L):
        W_T = jnp.concatenate([Wf[l, :D], Ws[l, :D]], axis=1)
        W_U = jnp.concatenate([Wf[l, D:2 * D], Ws[l, D:2 * D]], axis=1)
        W_E = jnp.concatenate([Wf[l, 2 * D:], Ws[l, 2 * D:]], axis=1)
        b_T = jnp.concatenate([bf[l], bs[l]]).reshape(1, 2 * D)
        if l == 0:
            x, T, U = _proj0(y, st, gamma, beta, W_T, W_U, b_T)
        else:
            x, T, U = _proju(aggr2, x, W_T, W_U, b_T)
        GT0, GU0 = _gather(T, U, dst, src, 0)
        m0 = _edge_stage(GT0, GU0, edge_attr, W_E, 0)
        GT1, GU1 = _gather(T, U, dst, src, 1)
        m1 = _edge_stage(GT1, GU1, edge_attr, W_E, 1)
        aggr2 = _scatter(m0, m1, dst)
    return _update(aggr2, x)


# speedup vs baseline: 6.0285x; 1.0770x over previous
import jax
import jax.numpy as jnp
from jax import lax
from jax.experimental import pallas as pl
from jax.experimental.pallas import tpu as pltpu
from jax.experimental.pallas import tpu_sc as plsc

N = 10000
E = 160000
D = 256
ED = 16

RB = 2000
EB = 2000

NC = 2
NS = 16
NW = NC * NS

GK = 128
GKG = 112
EH = E // 2
G_PER = EH // NW // 8 * 8
G_FULL = G_PER // GKG
G_LAST = G_PER - GKG
G_REM0 = NW * G_PER
G_REM1 = EH - GKG

S_PER = E // NS
S_FULL = S_PER // GK
S_TAIL = S_PER - S_FULL * GK

DH = D // 2
TROW = 624


def _mm_stats_body(x_ref, w_ref, b_ref, y_ref, st_ref):
    i = pl.program_id(0)
    y = jnp.dot(x_ref[...], w_ref[...], preferred_element_type=jnp.float32)
    y = y + b_ref[...]
    y_ref[...] = y
    s1 = jnp.sum(y, axis=0)
    s2 = jnp.sum(y * y, axis=0)
    rows = lax.broadcasted_iota(jnp.int32, (8, D), 0)
    upd = jnp.where(rows == 0, s1[None, :], 0.0) + jnp.where(rows == 1, s2[None, :], 0.0)

    @pl.when(i == 0)
    def _():
        st_ref[...] = jnp.zeros_like(st_ref)

    st_ref[...] += upd


def _mm_stats(node_attr, W0, b0):
    return pl.pallas_call(
        _mm_stats_body,
        grid=(N // RB,),
        in_specs=[
            pl.BlockSpec((RB, D), lambda i: (i, 0)),
            pl.BlockSpec((D, D), lambda i: (0, 0)),
            pl.BlockSpec((1, D), lambda i: (0, 0)),
        ],
        out_specs=[
            pl.BlockSpec((RB, D), lambda i: (i, 0)),
            pl.BlockSpec((8, D), lambda i: (0, 0)),
        ],
        out_shape=[
            jax.ShapeDtypeStruct((N, D), jnp.float32),
            jax.ShapeDtypeStruct((8, D), jnp.float32),
        ],
    )(node_attr, W0, b0.reshape(1, D))


def _pack_bf16_pair(lo, hi):
    ul = lax.bitcast_convert_type(lo, jnp.int32)
    uh = lax.bitcast_convert_type(hi, jnp.int32)
    ul = ul + 0x7FFF + ((ul >> 16) & 1)
    uh = uh + 0x7FFF + ((uh >> 16) & 1)
    return ((ul >> 16) & 0xFFFF) | (uh & jnp.int32(-65536))


def _unpack_bf16_pair(p):
    lo = lax.bitcast_convert_type(p << 16, jnp.float32)
    hi = lax.bitcast_convert_type(p & jnp.int32(-65536), jnp.float32)
    return lo, hi


def _proj_common(x, wt_ref, wu_ref, bt_ref, x_ref, t_ref, u_ref):
    x_ref[...] = x
    t = jnp.dot(x, wt_ref[...], preferred_element_type=jnp.float32) + bt_ref[...]
    u = jnp.dot(x, wu_ref[...], preferred_element_type=jnp.float32)
    t_ref[...] = _pack_bf16_pair(t[:, :D], t[:, D:])
    u_ref[...] = _pack_bf16_pair(u[:, :D], u[:, D:])


def _proj0_body(y_ref, st_ref, g_ref, be_ref, wt_ref, wu_ref, bt_ref,
                x_ref, t_ref, u_ref):
    mean = st_ref[0:1, :] / N
    var = st_ref[1:2, :] / N - mean * mean
    inv = g_ref[...] * lax.rsqrt(var + 1e-5)
    x = jnp.maximum((y_ref[...] - mean) * inv + be_ref[...], 0.0)
    _proj_common(x, wt_ref, wu_ref, bt_ref, x_ref, t_ref, u_ref)


def _proj0(y, st, gamma, beta, W_T, W_U, b_T):
    return pl.pallas_call(
        _proj0_body,
        grid=(N // RB,),
        in_specs=[
            pl.BlockSpec((RB, D), lambda i: (i, 0)),
            pl.BlockSpec((8, D), lambda i: (0, 0)),
            pl.BlockSpec((1, D), lambda i: (0, 0)),
            pl.BlockSpec((1, D), lambda i: (0, 0)),
            pl.BlockSpec((D, 2 * D), lambda i: (0, 0)),
            pl.BlockSpec((D, 2 * D), lambda i: (0, 0)),
            pl.BlockSpec((1, 2 * D), lambda i: (0, 0)),
        ],
        out_specs=[
            pl.BlockSpec((RB, D), lambda i: (i, 0)),
            pl.BlockSpec((RB, D), lambda i: (i, 0)),
            pl.BlockSpec((RB, D), lambda i: (i, 0)),
        ],
        out_shape=[
            jax.ShapeDtypeStruct((N, D), jnp.float32),
            jax.ShapeDtypeStruct((N, D), jnp.int32),
            jax.ShapeDtypeStruct((N, D), jnp.int32),
        ],
    )(y, st, gamma.reshape(1, D), beta.reshape(1, D), W_T, W_U, b_T)


def _proju_body(a_ref, xp_ref, wt_ref, wu_ref, bt_ref, x_ref, t_ref, u_ref):
    a = jnp.concatenate([a_ref[0], a_ref[1]], axis=1)
    x = jnp.maximum(a + xp_ref[...], 0.0)
    _proj_common(x, wt_ref, wu_ref, bt_ref, x_ref, t_ref, u_ref)


def _proju(aggr2, xp, W_T, W_U, b_T):
    return pl.pallas_call(
        _proju_body,
        grid=(N // RB,),
        in_specs=[
            pl.BlockSpec((2, RB, DH), lambda i: (0, i, 0)),
            pl.BlockSpec((RB, D), lambda i: (i, 0)),
            pl.BlockSpec((D, 2 * D), lambda i: (0, 0)),
            pl.BlockSpec((D, 2 * D), lambda i: (0, 0)),
            pl.BlockSpec((1, 2 * D), lambda i: (0, 0)),
        ],
        out_specs=[
            pl.BlockSpec((RB, D), lambda i: (i, 0)),
            pl.BlockSpec((RB, D), lambda i: (i, 0)),
            pl.BlockSpec((RB, D), lambda i: (i, 0)),
        ],
        out_shape=[
            jax.ShapeDtypeStruct((N, D), jnp.float32),
            jax.ShapeDtypeStruct((N, D), jnp.int32),
            jax.ShapeDtypeStruct((N, D), jnp.int32),
        ],
    )(aggr2, xp, W_T, W_U, b_T)


def _edge_body(gt_ref, gu_ref, ea_ref, we_ref, m_ref):
    ep = jnp.dot(ea_ref[...], we_ref[...], preferred_element_type=jnp.float32)
    ft, st = _unpack_bf16_pair(gt_ref[...])
    fu, su = _unpack_bf16_pair(gu_ref[...])
    f = ft + fu + ep[:, :D]
    s = st + su + ep[:, D:]
    sig = 1.0 / (1.0 + jnp.exp(-f))
    sp = jnp.maximum(s, 0.0) + jnp.log(1.0 + jnp.exp(-jnp.abs(s)))
    msg = sig * sp
    m_ref[...] = jnp.stack([msg[:, :DH], msg[:, DH:]], axis=0)


def _edge_stage(GT, GU, edge_attr, W_E, part):
    off = part * (EH // EB)
    return pl.pallas_call(
        _edge_body,
        grid=(EH // EB,),
        in_specs=[
            pl.BlockSpec((EB, D), lambda i: (i, 0)),
            pl.BlockSpec((EB, D), lambda i: (i, 0)),
            pl.BlockSpec((EB, ED), lambda i: (i + off, 0)),
            pl.BlockSpec((ED, 2 * D), lambda i: (0, 0)),
        ],
        out_specs=pl.BlockSpec((2, EB, DH), lambda i: (0, i, 0)),
        out_shape=jax.ShapeDtypeStruct((2, EH, DH), jnp.float32),
    )(GT, GU, edge_attr, W_E)


def _update_body(a_ref, x_ref, o_ref):
    a = jnp.concatenate([a_ref[0], a_ref[1]], axis=1)
    o_ref[...] = jnp.maximum(a + x_ref[...], 0.0)


def _update(aggr2, x):
    return pl.pallas_call(
        _update_body,
        grid=(N // RB,),
        in_specs=[
            pl.BlockSpec((2, RB, DH), lambda i: (0, i, 0)),
            pl.BlockSpec((RB, D), lambda i: (i, 0)),
        ],
        out_specs=pl.BlockSpec((RB, D), lambda i: (i, 0)),
        out_shape=jax.ShapeDtypeStruct((N, D), jnp.float32),
    )(aggr2, x)


def _make_gather_body(part):
    def body(t_hbm, u_hbm, dst_hbm, src_hbm, gt_hbm, gu_hbm,
             dbufs, sbufs, tbufs, ubufs, semt, semu, semwt, semwu):
        wid = lax.axis_index("s") * NC + lax.axis_index("c")
        base = wid * G_PER
        ibase = part * EH + base

        def start(j, b, wait_prev):
            off = base + j * GKG
            pltpu.sync_copy(dst_hbm.at[pl.ds(ibase + j * GKG, GKG)], dbufs[b])
            pltpu.sync_copy(src_hbm.at[pl.ds(ibase + j * GKG, GKG)], sbufs[b])
            if wait_prev:
                pltpu.make_async_copy(tbufs[b], gt_hbm.at[pl.ds(off, GKG)],
                                      semwt[b]).wait()
                pltpu.make_async_copy(ubufs[b], gu_hbm.at[pl.ds(off, GKG)],
                                      semwu[b]).wait()
            pltpu.async_copy(t_hbm.at[dbufs[b]], tbufs[b], semt[b])
            pltpu.async_copy(u_hbm.at[sbufs[b]], ubufs[b], semu[b])

        def finish(j, b):
            off = base + j * GKG
            pltpu.make_async_copy(t_hbm.at[dbufs[b]], tbufs[b], semt[b]).wait()
            pltpu.make_async_copy(u_hbm.at[sbufs[b]], ubufs[b], semu[b]).wait()
            pltpu.async_copy(tbufs[b], gt_hbm.at[pl.ds(off, GKG)], semwt[b])
            pltpu.async_copy(ubufs[b], gu_hbm.at[pl.ds(off, GKG)], semwu[b])

        start(0, 0, False)
        start(1, 1, False)

        def pair(p, _):
            j0 = 2 * p
            finish(j0, 0)
            start(j0 + 2, 0, True)
            finish(j0 + 1, 1)
            start(j0 + 3, 1, True)
            return 0

        lax.fori_loop(0, G_FULL // 2 - 1, pair, 0)
        finish(G_FULL - 2, 0)
        finish(G_FULL - 1, 1)

        off0 = base + G_LAST
        pltpu.make_async_copy(tbufs[0], gt_hbm.at[pl.ds(off0, GKG)], semwt[0]).wait()
        pltpu.make_async_copy(ubufs[0], gu_hbm.at[pl.ds(off0, GKG)], semwu[0]).wait()
        pltpu.make_async_copy(tbufs[1], gt_hbm.at[pl.ds(off0, GKG)], semwt[1]).wait()
        pltpu.make_async_copy(ubufs[1], gu_hbm.at[pl.ds(off0, GKG)], semwu[1]).wait()

        def sync_chunk(o):
            pltpu.sync_copy(dst_hbm.at[pl.ds(part * EH + o, GKG)], dbufs[0])
            pltpu.sync_copy(src_hbm.at[pl.ds(part * EH + o, GKG)], sbufs[0])
            pltpu.async_copy(t_hbm.at[dbufs[0]], tbufs[0], semt[0]).wait()
            pltpu.async_copy(u_hbm.at[sbufs[0]], ubufs[0], semu[0]).wait()
            pltpu.sync_copy(tbufs[0], gt_hbm.at[pl.ds(o, GKG)])
            pltpu.sync_copy(ubufs[0], gu_hbm.at[pl.ds(o, GKG)])

        sync_chunk(base + G_LAST)

        @pl.when(wid == 0)
        def _():
            sync_chunk(G_REM0)
            sync_chunk(G_REM1)

    return body


def _gather(T, U, dst, src, part):
    mesh = plsc.VectorSubcoreMesh(core_axis_name="c", subcore_axis_name="s")
    k = pl.kernel(
        _make_gather_body(part),
        out_type=[
            jax.ShapeDtypeStruct((EH, D), jnp.int32),
            jax.ShapeDtypeStruct((EH, D), jnp.int32),
        ],
        mesh=mesh,
        scratch_types=[
            [pltpu.VMEM((GKG,), jnp.int32)] * 2,
            [pltpu.VMEM((GKG,), jnp.int32)] * 2,
            [pltpu.VMEM((GKG, D), jnp.int32)] * 2,
            [pltpu.VMEM((GKG, D), jnp.int32)] * 2,
            [pltpu.SemaphoreType.DMA] * 2,
            [pltpu.SemaphoreType.DMA] * 2,
            [pltpu.SemaphoreType.DMA] * 2,
            [pltpu.SemaphoreType.DMA] * 2,
        ],
    )
    return k(T, U, dst, src)


def _scatter_body(m0_hbm, m1_hbm, dst_hbm, a_hbm,
                  spbuf, mbufs, ibufs, mbuf_t, ibuf_t, sems, semi):
    c = lax.axis_index("c")
    sid = lax.axis_index("s")
    mbuf = mbufs[0]

    def zrow(i, _):
        r = i // (DH // 16)
        k = i % (DH // 16)
        mbuf[r, pl.ds(k * 16, 16)] = jnp.zeros((16,), jnp.float32)
        return 0

    lax.fori_loop(0, GK * (DH // 16), zrow, 0)

    start = sid * TROW
    for t in range(TROW // GK):
        pltpu.sync_copy(mbuf, spbuf.at[pl.ds(start + t * GK, GK)])
    pltpu.sync_copy(mbuf.at[pl.ds(0, TROW - (TROW // GK) * GK)],
                    spbuf.at[pl.ds(start + (TROW // GK) * GK,
                                   TROW - (TROW // GK) * GK)])

    @pl.when(sid == NS - 1)
    def _():
        pltpu.sync_copy(mbuf.at[pl.ds(0, N - NS * TROW)],
                        spbuf.at[pl.ds(NS * TROW, N - NS * TROW)])

    plsc.subcore_barrier()

    base = sid * S_PER

    def run_part(m_hbm, lbase):
        def start(j, b):
            pltpu.async_copy(dst_hbm.at[pl.ds(base + j * GK, GK)], ibufs[b],
                             semi[b])
            pltpu.async_copy(m_hbm.at[c, pl.ds(lbase + j * GK, GK)],
                             mbufs[b], sems[b])

        def finish(j, b):
            pltpu.make_async_copy(dst_hbm.at[pl.ds(base + j * GK, GK)],
                                  ibufs[b], semi[b]).wait()
            pltpu.make_async_copy(m_hbm.at[c, pl.ds(lbase + j * GK, GK)],
                                  mbufs[b], sems[b]).wait()
            pltpu.sync_copy(mbufs[b], spbuf.at[ibufs[b]], add=True)

        start(0, 0)
        start(1, 1)

        def pair(p, _):
            j0 = 2 * p
            finish(j0, 0)
            start(j0 + 2, 0)
            finish(j0 + 1, 1)
            start(j0 + 3, 1)
            return 0

        lax.fori_loop(0, S_FULL // 2 - 1, pair, 0)
        finish(S_FULL - 2, 0)
        finish(S_FULL - 1, 1)

        pltpu.sync_copy(dst_hbm.at[pl.ds(base + S_FULL * GK, S_TAIL)], ibuf_t)
        pltpu.sync_copy(m_hbm.at[c, pl.ds(lbase + S_FULL * GK, S_TAIL)], mbuf_t)
        pltpu.sync_copy(mbuf_t, spbuf.at[ibuf_t], add=True)

    @pl.when(sid < NS // 2)
    def _():
        run_part(m0_hbm, sid * S_PER)

    @pl.when(sid >= NS // 2)
    def _():
        run_part(m1_hbm, sid * S_PER - EH)

    plsc.subcore_barrier()
    pltpu.sync_copy(spbuf.at[pl.ds(sid * TROW, TROW)],
                    a_hbm.at[c, pl.ds(sid * TROW, TROW)])

    @pl.when(sid == NS - 1)
    def _():
        pltpu.sync_copy(spbuf.at[pl.ds(NS * TROW, N - NS * TROW)],
                        a_hbm.at[c, pl.ds(NS * TROW, N - NS * TROW)])


def _scatter(m0, m1, dst):
    mesh = plsc.VectorSubcoreMesh(core_axis_name="c", subcore_axis_name="s")
    k = pl.kernel(
        _scatter_body,
        out_type=jax.ShapeDtypeStruct((2, N, DH), jnp.float32),
        mesh=mesh,
        scratch_types=[
            pltpu.VMEM_SHARED((N, DH), jnp.float32),
            [pltpu.VMEM((GK, DH), jnp.float32)] * 2,
            [pltpu.VMEM((GK,), jnp.int32)] * 2,
            pltpu.VMEM((S_TAIL, DH), jnp.float32),
            pltpu.VMEM((S_TAIL,), jnp.int32),
            [pltpu.SemaphoreType.DMA] * 2,
            [pltpu.SemaphoreType.DMA] * 2,
        ],
    )
    return k(m0, m1, dst)


def kernel(node_attr, edge_index, edge_attr, W0, b0, gamma, beta, Wf, bf, Ws, bs):
    src = edge_index[0]
    dst = edge_index[1]

    y, st = _mm_stats(node_attr, W0, b0)

    L = Wf.shape[0]
    x = None
    aggr2 = None
    for l in range(L):
        W_T = jnp.concatenate([Wf[l, :D], Ws[l, :D]], axis=1)
        W_U = jnp.concatenate([Wf[l, D:2 * D], Ws[l, D:2 * D]], axis=1)
        W_E = jnp.concatenate([Wf[l, 2 * D:], Ws[l, 2 * D:]], axis=1)
        b_T = jnp.concatenate([bf[l], bs[l]]).reshape(1, 2 * D)
        if l == 0:
            x, T, U = _proj0(y, st, gamma, beta, W_T, W_U, b_T)
        else:
            x, T, U = _proju(aggr2, x, W_T, W_U, b_T)
        GT0, GU0 = _gather(T, U, dst, src, 0)
        m0 = _edge_stage(GT0, GU0, edge_attr, W_E, 0)
        GT1, GU1 = _gather(T, U, dst, src, 1)
        m1 = _edge_stage(GT1, GU1, edge_attr, W_E, 1)
        aggr2 = _scatter(m0, m1, dst)
    return _update(aggr2, x)


# speedup vs baseline: 6.0931x; 1.0107x over previous
import jax
import jax.numpy as jnp
from jax import lax
from jax.experimental import pallas as pl
from jax.experimental.pallas import tpu as pltpu
from jax.experimental.pallas import tpu_sc as plsc

N = 10000
E = 160000
D = 256
ED = 16

RB = 2000
EB = 4000

NC = 2
NS = 16
NW = NC * NS

GK = 128
GKG = 112
EH = E // 2
G_PER = EH // NW // 8 * 8
G_FULL = G_PER // GKG
G_LAST = G_PER - GKG
G_REM0 = NW * G_PER
G_REM1 = EH - GKG

S_PER = E // NS
S_FULL = S_PER // GK
S_TAIL = S_PER - S_FULL * GK

DH = D // 2
TROW = 624


def _mm_stats_body(x_ref, w_ref, b_ref, y_ref, st_ref):
    i = pl.program_id(0)
    y = jnp.dot(x_ref[...], w_ref[...], preferred_element_type=jnp.float32)
    y = y + b_ref[...]
    y_ref[...] = y
    s1 = jnp.sum(y, axis=0)
    s2 = jnp.sum(y * y, axis=0)
    rows = lax.broadcasted_iota(jnp.int32, (8, D), 0)
    upd = jnp.where(rows == 0, s1[None, :], 0.0) + jnp.where(rows == 1, s2[None, :], 0.0)

    @pl.when(i == 0)
    def _():
        st_ref[...] = jnp.zeros_like(st_ref)

    st_ref[...] += upd


def _mm_stats(node_attr, W0, b0):
    return pl.pallas_call(
        _mm_stats_body,
        grid=(N // RB,),
        in_specs=[
            pl.BlockSpec((RB, D), lambda i: (i, 0)),
            pl.BlockSpec((D, D), lambda i: (0, 0)),
            pl.BlockSpec((1, D), lambda i: (0, 0)),
        ],
        out_specs=[
            pl.BlockSpec((RB, D), lambda i: (i, 0)),
            pl.BlockSpec((8, D), lambda i: (0, 0)),
        ],
        out_shape=[
            jax.ShapeDtypeStruct((N, D), jnp.float32),
            jax.ShapeDtypeStruct((8, D), jnp.float32),
        ],
    )(node_attr, W0, b0.reshape(1, D))


def _pack_bf16_pair(lo, hi):
    ul = lax.bitcast_convert_type(lo, jnp.int32)
    uh = lax.bitcast_convert_type(hi, jnp.int32)
    ul = ul + 0x7FFF + ((ul >> 16) & 1)
    uh = uh + 0x7FFF + ((uh >> 16) & 1)
    return ((ul >> 16) & 0xFFFF) | (uh & jnp.int32(-65536))


def _unpack_bf16_pair(p):
    lo = lax.bitcast_convert_type(p << 16, jnp.float32)
    hi = lax.bitcast_convert_type(p & jnp.int32(-65536), jnp.float32)
    return lo, hi


def _proj_common(x, wt_ref, wu_ref, bt_ref, x_ref, t_ref, u_ref):
    x_ref[...] = x
    t = jnp.dot(x, wt_ref[...], preferred_element_type=jnp.float32) + bt_ref[...]
    u = jnp.dot(x, wu_ref[...], preferred_element_type=jnp.float32)
    t_ref[...] = _pack_bf16_pair(t[:, :D], t[:, D:])
    u_ref[...] = _pack_bf16_pair(u[:, :D], u[:, D:])


def _proj0_body(y_ref, st_ref, g_ref, be_ref, wt_ref, wu_ref, bt_ref,
                x_ref, t_ref, u_ref):
    mean = st_ref[0:1, :] / N
    var = st_ref[1:2, :] / N - mean * mean
    inv = g_ref[...] * lax.rsqrt(var + 1e-5)
    x = jnp.maximum((y_ref[...] - mean) * inv + be_ref[...], 0.0)
    _proj_common(x, wt_ref, wu_ref, bt_ref, x_ref, t_ref, u_ref)


def _proj0(y, st, gamma, beta, W_T, W_U, b_T):
    return pl.pallas_call(
        _proj0_body,
        grid=(N // RB,),
        in_specs=[
            pl.BlockSpec((RB, D), lambda i: (i, 0)),
            pl.BlockSpec((8, D), lambda i: (0, 0)),
            pl.BlockSpec((1, D), lambda i: (0, 0)),
            pl.BlockSpec((1, D), lambda i: (0, 0)),
            pl.BlockSpec((D, 2 * D), lambda i: (0, 0)),
            pl.BlockSpec((D, 2 * D), lambda i: (0, 0)),
            pl.BlockSpec((1, 2 * D), lambda i: (0, 0)),
        ],
        out_specs=[
            pl.BlockSpec((RB, D), lambda i: (i, 0)),
            pl.BlockSpec((RB, D), lambda i: (i, 0)),
            pl.BlockSpec((RB, D), lambda i: (i, 0)),
        ],
        out_shape=[
            jax.ShapeDtypeStruct((N, D), jnp.float32),
            jax.ShapeDtypeStruct((N, D), jnp.int32),
            jax.ShapeDtypeStruct((N, D), jnp.int32),
        ],
    )(y, st, gamma.reshape(1, D), beta.reshape(1, D), W_T, W_U, b_T)


def _proju_body(a_ref, xp_ref, wt_ref, wu_ref, bt_ref, x_ref, t_ref, u_ref):
    a = jnp.concatenate([a_ref[0], a_ref[1]], axis=1)
    x = jnp.maximum(a + xp_ref[...], 0.0)
    _proj_common(x, wt_ref, wu_ref, bt_ref, x_ref, t_ref, u_ref)


def _proju(aggr2, xp, W_T, W_U, b_T):
    return pl.pallas_call(
        _proju_body,
        grid=(N // RB,),
        in_specs=[
            pl.BlockSpec((2, RB, DH), lambda i: (0, i, 0)),
            pl.BlockSpec((RB, D), lambda i: (i, 0)),
            pl.BlockSpec((D, 2 * D), lambda i: (0, 0)),
            pl.BlockSpec((D, 2 * D), lambda i: (0, 0)),
            pl.BlockSpec((1, 2 * D), lambda i: (0, 0)),
        ],
        out_specs=[
            pl.BlockSpec((RB, D), lambda i: (i, 0)),
            pl.BlockSpec((RB, D), lambda i: (i, 0)),
            pl.BlockSpec((RB, D), lambda i: (i, 0)),
        ],
        out_shape=[
            jax.ShapeDtypeStruct((N, D), jnp.float32),
            jax.ShapeDtypeStruct((N, D), jnp.int32),
            jax.ShapeDtypeStruct((N, D), jnp.int32),
        ],
    )(aggr2, xp, W_T, W_U, b_T)


def _edge_body(gt_ref, gu_ref, ea_ref, we_ref, m_ref):
    ep = jnp.dot(ea_ref[...], we_ref[...], preferred_element_type=jnp.float32)
    ft, st = _unpack_bf16_pair(gt_ref[...])
    fu, su = _unpack_bf16_pair(gu_ref[...])
    f = ft + fu + ep[:, :D]
    s = st + su + ep[:, D:]
    sig = 1.0 / (1.0 + jnp.exp(-f))
    sp = jnp.maximum(s, 0.0) + jnp.log(1.0 + jnp.exp(-jnp.abs(s)))
    msg = sig * sp
    m_ref[...] = jnp.stack([msg[:, :DH], msg[:, DH:]], axis=0)


def _edge_stage(GT, GU, edge_attr, W_E, part):
    off = part * (EH // EB)
    return pl.pallas_call(
        _edge_body,
        grid=(EH // EB,),
        in_specs=[
            pl.BlockSpec((EB, D), lambda i: (i, 0)),
            pl.BlockSpec((EB, D), lambda i: (i, 0)),
            pl.BlockSpec((EB, ED), lambda i: (i + off, 0)),
            pl.BlockSpec((ED, 2 * D), lambda i: (0, 0)),
        ],
        out_specs=pl.BlockSpec((2, EB, DH), lambda i: (0, i, 0)),
        out_shape=jax.ShapeDtypeStruct((2, EH, DH), jnp.float32),
    )(GT, GU, edge_attr, W_E)


def _update_body(a_ref, x_ref, o_ref):
    a = jnp.concatenate([a_ref[0], a_ref[1]], axis=1)
    o_ref[...] = jnp.maximum(a + x_ref[...], 0.0)


def _update(aggr2, x):
    return pl.pallas_call(
        _update_body,
        grid=(N // RB,),
        in_specs=[
            pl.BlockSpec((2, RB, DH), lambda i: (0, i, 0)),
            pl.BlockSpec((RB, D), lambda i: (i, 0)),
        ],
        out_specs=pl.BlockSpec((RB, D), lambda i: (i, 0)),
        out_shape=jax.ShapeDtypeStruct((N, D), jnp.float32),
    )(aggr2, x)


def _make_gather_body(part):
    def body(t_hbm, u_hbm, dst_hbm, src_hbm, gt_hbm, gu_hbm,
             dbufs, sbufs, tbufs, ubufs, semt, semu, semwt, semwu):
        wid = lax.axis_index("s") * NC + lax.axis_index("c")
        base = wid * G_PER
        ibase = part * EH + base

        def start(j, b, wait_prev):
            off = base + j * GKG
            pltpu.sync_copy(dst_hbm.at[pl.ds(ibase + j * GKG, GKG)], dbufs[b])
            pltpu.sync_copy(src_hbm.at[pl.ds(ibase + j * GKG, GKG)], sbufs[b])
            if wait_prev:
                pltpu.make_async_copy(tbufs[b], gt_hbm.at[pl.ds(off, GKG)],
                                      semwt[b]).wait()
                pltpu.make_async_copy(ubufs[b], gu_hbm.at[pl.ds(off, GKG)],
                                      semwu[b]).wait()
            pltpu.async_copy(t_hbm.at[dbufs[b]], tbufs[b], semt[b])
            pltpu.async_copy(u_hbm.at[sbufs[b]], ubufs[b], semu[b])

        def finish(j, b):
            off = base + j * GKG
            pltpu.make_async_copy(t_hbm.at[dbufs[b]], tbufs[b], semt[b]).wait()
            pltpu.make_async_copy(u_hbm.at[sbufs[b]], ubufs[b], semu[b]).wait()
            pltpu.async_copy(tbufs[b], gt_hbm.at[pl.ds(off, GKG)], semwt[b])
            pltpu.async_copy(ubufs[b], gu_hbm.at[pl.ds(off, GKG)], semwu[b])

        start(0, 0, False)
        start(1, 1, False)

        def pair(p, _):
            j0 = 2 * p
            finish(j0, 0)
            start(j0 + 2, 0, True)
            finish(j0 + 1, 1)
            start(j0 + 3, 1, True)
            return 0

        lax.fori_loop(0, G_FULL // 2 - 1, pair, 0)
        finish(G_FULL - 2, 0)
        finish(G_FULL - 1, 1)

        off0 = base + G_LAST
        pltpu.make_async_copy(tbufs[0], gt_hbm.at[pl.ds(off0, GKG)], semwt[0]).wait()
        pltpu.make_async_copy(ubufs[0], gu_hbm.at[pl.ds(off0, GKG)], semwu[0]).wait()
        pltpu.make_async_copy(tbufs[1], gt_hbm.at[pl.ds(off0, GKG)], semwt[1]).wait()
        pltpu.make_async_copy(ubufs[1], gu_hbm.at[pl.ds(off0, GKG)], semwu[1]).wait()

        def sync_chunk(o):
            pltpu.sync_copy(dst_hbm.at[pl.ds(part * EH + o, GKG)], dbufs[0])
            pltpu.sync_copy(src_hbm.at[pl.ds(part * EH + o, GKG)], sbufs[0])
            pltpu.async_copy(t_hbm.at[dbufs[0]], tbufs[0], semt[0]).wait()
            pltpu.async_copy(u_hbm.at[sbufs[0]], ubufs[0], semu[0]).wait()
            pltpu.sync_copy(tbufs[0], gt_hbm.at[pl.ds(o, GKG)])
            pltpu.sync_copy(ubufs[0], gu_hbm.at[pl.ds(o, GKG)])

        sync_chunk(base + G_LAST)

        @pl.when(wid == 0)
        def _():
            sync_chunk(G_REM0)
            sync_chunk(G_REM1)

    return body


def _gather(T, U, dst, src, part):
    mesh = plsc.VectorSubcoreMesh(core_axis_name="c", subcore_axis_name="s")
    k = pl.kernel(
        _make_gather_body(part),
        out_type=[
            jax.ShapeDtypeStruct((EH, D), jnp.int32),
            jax.ShapeDtypeStruct((EH, D), jnp.int32),
        ],
        mesh=mesh,
        scratch_types=[
            [pltpu.VMEM((GKG,), jnp.int32)] * 2,
            [pltpu.VMEM((GKG,), jnp.int32)] * 2,
            [pltpu.VMEM((GKG, D), jnp.int32)] * 2,
            [pltpu.VMEM((GKG, D), jnp.int32)] * 2,
            [pltpu.SemaphoreType.DMA] * 2,
            [pltpu.SemaphoreType.DMA] * 2,
            [pltpu.SemaphoreType.DMA] * 2,
            [pltpu.SemaphoreType.DMA] * 2,
        ],
    )
    return k(T, U, dst, src)


def _scatter_body(m0_hbm, m1_hbm, dst_hbm, a_hbm,
                  spbuf, mbufs, ibufs, mbuf_t, ibuf_t, sems, semi):
    c = lax.axis_index("c")
    sid = lax.axis_index("s")
    mbuf = mbufs[0]

    def zrow(i, _):
        r = i // (DH // 16)
        k = i % (DH // 16)
        mbuf[r, pl.ds(k * 16, 16)] = jnp.zeros((16,), jnp.float32)
        return 0

    lax.fori_loop(0, GK * (DH // 16), zrow, 0)

    start = sid * TROW
    for t in range(TROW // GK):
        pltpu.sync_copy(mbuf, spbuf.at[pl.ds(start + t * GK, GK)])
    pltpu.sync_copy(mbuf.at[pl.ds(0, TROW - (TROW // GK) * GK)],
                    spbuf.at[pl.ds(start + (TROW // GK) * GK,
                                   TROW - (TROW // GK) * GK)])

    @pl.when(sid == NS - 1)
    def _():
        pltpu.sync_copy(mbuf.at[pl.ds(0, N - NS * TROW)],
                        spbuf.at[pl.ds(NS * TROW, N - NS * TROW)])

    plsc.subcore_barrier()

    base = sid * S_PER

    def run_part(m_hbm, lbase):
        def start(j, b):
            pltpu.async_copy(dst_hbm.at[pl.ds(base + j * GK, GK)], ibufs[b],
                             semi[b])
            pltpu.async_copy(m_hbm.at[c, pl.ds(lbase + j * GK, GK)],
                             mbufs[b], sems[b])

        def finish(j, b):
            pltpu.make_async_copy(dst_hbm.at[pl.ds(base + j * GK, GK)],
                                  ibufs[b], semi[b]).wait()
            pltpu.make_async_copy(m_hbm.at[c, pl.ds(lbase + j * GK, GK)],
                                  mbufs[b], sems[b]).wait()
            pltpu.sync_copy(mbufs[b], spbuf.at[ibufs[b]], add=True)

        start(0, 0)
        start(1, 1)

        def pair(p, _):
            j0 = 2 * p
            finish(j0, 0)
            start(j0 + 2, 0)
            finish(j0 + 1, 1)
            start(j0 + 3, 1)
            return 0

        lax.fori_loop(0, S_FULL // 2 - 1, pair, 0)
        finish(S_FULL - 2, 0)
        finish(S_FULL - 1, 1)

        pltpu.sync_copy(dst_hbm.at[pl.ds(base + S_FULL * GK, S_TAIL)], ibuf_t)
        pltpu.sync_copy(m_hbm.at[c, pl.ds(lbase + S_FULL * GK, S_TAIL)], mbuf_t)
        pltpu.sync_copy(mbuf_t, spbuf.at[ibuf_t], add=True)

    @pl.when(sid < NS // 2)
    def _():
        run_part(m0_hbm, sid * S_PER)

    @pl.when(sid >= NS // 2)
    def _():
        run_part(m1_hbm, sid * S_PER - EH)

    plsc.subcore_barrier()
    pltpu.sync_copy(spbuf.at[pl.ds(sid * TROW, TROW)],
                    a_hbm.at[c, pl.ds(sid * TROW, TROW)])

    @pl.when(sid == NS - 1)
    def _():
        pltpu.sync_copy(spbuf.at[pl.ds(NS * TROW, N - NS * TROW)],
                        a_hbm.at[c, pl.ds(NS * TROW, N - NS * TROW)])


def _scatter(m0, m1, dst):
    mesh = plsc.VectorSubcoreMesh(core_axis_name="c", subcore_axis_name="s")
    k = pl.kernel(
        _scatter_body,
        out_type=jax.ShapeDtypeStruct((2, N, DH), jnp.float32),
        mesh=mesh,
        scratch_types=[
            pltpu.VMEM_SHARED((N, DH), jnp.float32),
            [pltpu.VMEM((GK, DH), jnp.float32)] * 2,
            [pltpu.VMEM((GK,), jnp.int32)] * 2,
            pltpu.VMEM((S_TAIL, DH), jnp.float32),
            pltpu.VMEM((S_TAIL,), jnp.int32),
            [pltpu.SemaphoreType.DMA] * 2,
            [pltpu.SemaphoreType.DMA] * 2,
        ],
    )
    return k(m0, m1, dst)


def kernel(node_attr, edge_index, edge_attr, W0, b0, gamma, beta, Wf, bf, Ws, bs):
    src = edge_index[0]
    dst = edge_index[1]

    y, st = _mm_stats(node_attr, W0, b0)

    L = Wf.shape[0]
    x = None
    aggr2 = None
    for l in range(L):
        W_T = jnp.concatenate([Wf[l, :D], Ws[l, :D]], axis=1)
        W_U = jnp.concatenate([Wf[l, D:2 * D], Ws[l, D:2 * D]], axis=1)
        W_E = jnp.concatenate([Wf[l, 2 * D:], Ws[l, 2 * D:]], axis=1)
        b_T = jnp.concatenate([bf[l], bs[l]]).reshape(1, 2 * D)
        if l == 0:
            x, T, U = _proj0(y, st, gamma, beta, W_T, W_U, b_T)
        else:
            x, T, U = _proju(aggr2, x, W_T, W_U, b_T)
        GT0, GU0 = _gather(T, U, dst, src, 0)
        m0 = _edge_stage(GT0, GU0, edge_attr, W_E, 0)
        GT1, GU1 = _gather(T, U, dst, src, 1)
        m1 = _edge_stage(GT1, GU1, edge_attr, W_E, 1)
        aggr2 = _scatter(m0, m1, dst)
    return _update(aggr2, x)
